# baseline (speedup 1.0000x reference)
import numpy as np
import jax
import jax.numpy as jnp
from jax.experimental import pallas as pl

N = 10000
E = 160000
TOWERS = 5
F_IN = 80
F_OUT = 16
NUM_GRAPHS = 64
AVG_DEG_LOG = float(np.log(17.0))


def _identity_pallas(x):
    def body(x_ref, o_ref):
        o_ref[...] = x_ref[...]
    return pl.pallas_call(
        body, out_shape=jax.ShapeDtypeStruct(x.shape, x.dtype))(x)


def _pna_conv(x, src, dst, pre_w, pre_b, post_w, post_b, lw, lb):
    h = jnp.concatenate([x[dst], x[src]], axis=-1)
    m = jnp.einsum('ei,tio->eto', h, pre_w) + pre_b
    cnt = jax.ops.segment_sum(jnp.ones((m.shape[0],), jnp.float32), dst, num_segments=N)
    deg = jnp.maximum(cnt, 1.0)[:, None, None]
    s = jax.ops.segment_sum(m, dst, num_segments=N)
    mean = s / deg
    mean_sq = jax.ops.segment_sum(m * m, dst, num_segments=N) / deg
    std = jnp.sqrt(jnp.maximum(mean_sq - mean * mean, 0.0) + 1e-5)
    has = (cnt > 0)[:, None, None]
    mn = jnp.where(has, jax.ops.segment_min(m, dst, num_segments=N), 0.0)
    mx = jnp.where(has, jax.ops.segment_max(m, dst, num_segments=N), 0.0)
    agg = jnp.concatenate([mean, mn, mx, std], axis=-1)
    d = jnp.maximum(cnt, 1.0)
    amp = (jnp.log(d + 1.0) / AVG_DEG_LOG)[:, None, None]
    att = (AVG_DEG_LOG / jnp.log(d + 1.0))[:, None, None]
    out = jnp.concatenate([agg, agg * amp, agg * att], axis=-1)
    xt = jnp.broadcast_to(x[:, None, :], (N, TOWERS, F_IN))
    out = jnp.concatenate([xt, out], axis=-1)
    out = jnp.einsum('nti,tio->nto', out, post_w) + post_b
    out = out.reshape(N, TOWERS * F_OUT)
    return out @ lw + lb


def _batch_norm(x, g, b):
    mu = x.mean(axis=0)
    var = x.var(axis=0)
    return (x - mu) / jnp.sqrt(var + 1e-5) * g + b


def kernel(x, edge_index, batch, node_emb, pre_w1, pre_b1, post_w1, post_b1,
           conv_lin_w1, conv_lin_b1, bn_g1, bn_b1,
           pre_w2, pre_b2, post_w2, post_b2, conv_lin_w2, conv_lin_b2, bn_g2, bn_b2,
           lin_w, lin_b):
    src, dst = edge_index[0], edge_index[1]
    h = node_emb[x].reshape(-1, F_IN)
    h = _identity_pallas(h)
    h = jax.nn.relu(_batch_norm(_pna_conv(h, src, dst, pre_w1, pre_b1, post_w1, post_b1, conv_lin_w1, conv_lin_b1), bn_g1, bn_b1))
    h = jax.nn.relu(_batch_norm(_pna_conv(h, src, dst, pre_w2, pre_b2, post_w2, post_b2, conv_lin_w2, conv_lin_b2), bn_g2, bn_b2))
    pooled = jax.ops.segment_max(h, batch, num_segments=NUM_GRAPHS)
    gcnt = jax.ops.segment_sum(jnp.ones((N,), jnp.float32), batch, num_segments=NUM_GRAPHS)
    pooled = jnp.where((gcnt > 0)[:, None], pooled, 0.0)
    return pooled @ lin_w + lin_b


# speedup vs baseline: 26.0477x; 26.0477x over previous
import functools
import numpy as np
import jax
import jax.numpy as jnp
from jax import lax
from jax.experimental import pallas as pl
from jax.experimental.pallas import tpu as pltpu
from jax.experimental.pallas import tpu_sc as plsc

N = 10000
E = 160000
TOWERS = 5
F_IN = 80
F_OUT = 16
NUM_GRAPHS = 64
VOCAB = 10000
AVG_DEG_LOG = float(np.log(17.0))

NW = 32
NP = 320
NPR = NP + 8
NPAD = NW * NP
FCH = 64
NFP = 7
FP = NFP * FCH
ECH = 2000
LCAP = E + 4096
GCH = 128
SCH = 512

@functools.lru_cache(maxsize=None)
def _mesh():
    return plsc.VectorSubcoreMesh(core_axis_name="c", subcore_axis_name="s")


def _wid():
    return lax.axis_index("s") * 2 + lax.axis_index("c")


NIDX = N * 40
NGCH = NIDX // GCH


@functools.lru_cache(maxsize=None)
def _emb_kernel():
    @functools.partial(
        pl.kernel, mesh=_mesh(),
        compiler_params=pltpu.CompilerParams(use_tc_tiling_on_sc=False, needs_layout_passes=False),
        name="sc_emb",
        out_type=jax.ShapeDtypeStruct((NIDX, 16), jnp.float32),
        scratch_types=[
            pltpu.VMEM((GCH,), jnp.int32),
            pltpu.VMEM((GCH, 16), jnp.float32),
            pltpu.SemaphoreType.DMA,
        ],
    )
    def k(table_hbm, idx_hbm, out_hbm, idx_v, rows_v, sem):
        w = _wid()
        nci = (NGCH - w + NW - 1) // NW

        def body(c, carry):
            ci = w + c * NW
            base = pl.multiple_of(ci * GCH, GCH)
            pltpu.sync_copy(idx_hbm.at[pl.ds(base, GCH)], idx_v)
            pltpu.async_copy(table_hbm.at[idx_v], rows_v, sem).wait()
            pltpu.sync_copy(rows_v, out_hbm.at[pl.ds(base, GCH)])
            return carry

        lax.fori_loop(0, nci, body, 0)
    return k


def _sc_emb(table16, idx):
    return _emb_kernel()(table16, idx)


@functools.lru_cache(maxsize=None)
def _csr_kernel():
    @functools.partial(
        pl.kernel, mesh=_mesh(),
        compiler_params=pltpu.CompilerParams(use_tc_tiling_on_sc=False, needs_layout_passes=False),
        name="sc_csr",
        out_type=[
            jax.ShapeDtypeStruct((NW, LCAP), jnp.int32),
            jax.ShapeDtypeStruct((NW, LCAP), jnp.int32),
            jax.ShapeDtypeStruct((NPAD,), jnp.float32),
            jax.ShapeDtypeStruct((NW * 8,), jnp.int32),
        ],
        scratch_types=[
            pltpu.VMEM((ECH,), jnp.int32),
            pltpu.VMEM((ECH,), jnp.int32),
            pltpu.VMEM((ECH + 16,), jnp.int32),
            pltpu.VMEM((ECH + 16,), jnp.int32),
            pltpu.VMEM((NP + 16,), jnp.float32),
            pltpu.VMEM((16,), jnp.int32),
        ],
    )
    def k(edge_hbm, srcs_hbm, ldst_hbm, cnt_hbm, tcnt_hbm,
          sv, dv, csv, clv, cntv, tmpv):
        _csr_body(edge_hbm, srcs_hbm, ldst_hbm, cnt_hbm, tcnt_hbm,
                  sv, dv, csv, clv, cntv, tmpv)
    return k


def _sc_csr(edge_index):
    return _csr_kernel()(edge_index)


def _csr_body(edge_hbm, srcs_hbm, ldst_hbm, cnt_hbm, tcnt_hbm,
              sv, dv, csv, clv, cntv, tmpv):
    w = _wid()
    lo = pl.multiple_of(w * NP, NP)

    def zbody(i, carry):
        cntv[pl.ds(i * 16, 16)] = jnp.zeros((16,), jnp.float32)
        return carry
    lax.fori_loop(0, (NP + 16) // 16, zbody, 0)

    lanes = lax.iota(jnp.int32, 16)
    filler_ld = NP + (lanes % 8)

    def chunk(c, w_off):
        pltpu.sync_copy(edge_hbm.at[0, pl.ds(c * ECH, ECH)], sv)
        pltpu.sync_copy(edge_hbm.at[1, pl.ds(c * ECH, ECH)], dv)

        def step(j, cur):
            d = dv[pl.ds(j * 16, 16)]
            s = sv[pl.ds(j * 16, 16)]
            m = (d >= lo) & (d < lo + NP)
            ld = jnp.where(m, d - lo, NP)
            rank = plsc.cumsum(jnp.where(m, 1, 0).astype(jnp.int32))
            pos = jnp.where(m, cur + rank - 1, cur + 15)
            plsc.store_scatter(csv, [pos], s)
            plsc.store_scatter(clv, [pos], ld)
            plsc.addupdate_scatter(cntv, [ld], jnp.ones((16,), jnp.float32))
            return cur + rank[15]

        cur = lax.fori_loop(0, ECH // 16, step, 0)
        csv[pl.ds(cur, 16)] = jnp.full((16,), w, jnp.int32)
        clv[pl.ds(cur, 16)] = filler_ld
        cur_pad = ((cur + 15) // 16) * 16
        w_off = pl.multiple_of(w_off, 16)
        pltpu.sync_copy(csv, srcs_hbm.at[w, pl.ds(w_off, ECH + 16)])
        pltpu.sync_copy(clv, ldst_hbm.at[w, pl.ds(w_off, ECH + 16)])
        return w_off + cur_pad

    total = lax.fori_loop(0, E // ECH, chunk, 0)
    tmpv[...] = jnp.broadcast_to(total, (16,)).astype(jnp.int32)
    pltpu.sync_copy(tmpv.at[pl.ds(0, 8)],
                    tcnt_hbm.at[pl.ds(pl.multiple_of(w * 8, 8), 8)])
    pltpu.sync_copy(cntv.at[pl.ds(0, NP)], cnt_hbm.at[pl.ds(lo, NP)])


BIGF = 3.0e38


@functools.lru_cache(maxsize=None)
def _segred_kernel():
    @functools.partial(
        pl.kernel, mesh=_mesh(),
        compiler_params=pltpu.CompilerParams(use_tc_tiling_on_sc=False, needs_layout_passes=False),
        name="sc_segred",
        out_type=jax.ShapeDtypeStruct((4, NW * NPR, FP), jnp.float32),
        scratch_types=[
            pltpu.VMEM((SCH,), jnp.int32),
            pltpu.VMEM((SCH + 16,), jnp.int32),
            pltpu.VMEM((4, GCH), jnp.int32),
            pltpu.VMEM((SCH, FCH), jnp.float32),
            pltpu.VMEM((NPR, FCH), jnp.float32),
            pltpu.VMEM((NPR, FCH), jnp.float32),
            pltpu.VMEM((NPR, FCH), jnp.float32),
            pltpu.VMEM((NPR, FCH), jnp.float32),
            pltpu.VMEM((16,), jnp.int32),
            pltpu.SemaphoreType.DMA,
        ],
    )
    def k(b7_hbm, srcs_hbm, ldst_hbm, tcnt_hbm, out_hbm,
          sv, lv, idxv, rows, st_s, st_q, st_mn, st_mx, ntv, sem):
        _segred_body(b7_hbm, srcs_hbm, ldst_hbm, tcnt_hbm, out_hbm,
                     sv, lv, idxv, rows, st_s, st_q, st_mn, st_mx, ntv, sem)
    return k


def _sc_segred(b7, srcs, ldsts, tcnt):
    return _segred_kernel()(b7, srcs, ldsts, tcnt)


def _segred_body(b7_hbm, srcs_hbm, ldst_hbm, tcnt_hbm, out_hbm,
                 sv, lv, idxv, rows, st_s, st_q, st_mn, st_mx, ntv, sem):
    w = _wid()
    pltpu.sync_copy(tcnt_hbm.at[pl.ds(pl.multiple_of(w * 8, 8), 8)],
                    ntv.at[pl.ds(0, 8)])
    nt = ntv[pl.ds(0, 16)][0]
    nchunks = (nt + SCH - 1) // SCH
    lanes = lax.iota(jnp.int32, 16)

    def fpass(k, carry0):
        def zrow(i, carry):
            for j in range(FCH // 16):
                sl = pl.ds(j * 16, 16)
                st_s[i, sl] = jnp.zeros((16,), jnp.float32)
                st_q[i, sl] = jnp.zeros((16,), jnp.float32)
                st_mn[i, sl] = jnp.full((16,), BIGF, jnp.float32)
                st_mx[i, sl] = jnp.full((16,), -BIGF, jnp.float32)
            return carry
        lax.fori_loop(0, NPR, zrow, 0)

        kbase = k * N

        def chunk(c, carry):
            off = pl.multiple_of(c * SCH, SCH)
            pltpu.sync_copy(srcs_hbm.at[w, pl.ds(off, SCH)], sv)
            pltpu.sync_copy(ldst_hbm.at[w, pl.ds(off, SCH)],
                            lv.at[pl.ds(0, SCH)])
            def mkidx(j, carry2):
                pos = off + j * 16 + lanes
                s = sv[pl.ds(j * 16, 16)]
                s = jnp.where(pos < nt, s, w)
                idxv[j // (GCH // 16), pl.ds((j % (GCH // 16)) * 16, 16)] = (
                    s + kbase)
                return carry2
            lax.fori_loop(0, SCH // 16, mkidx, 0)
            for q in range(SCH // GCH):
                pltpu.async_copy(b7_hbm.at[idxv.at[q]],
                                 rows.at[pl.ds(q * GCH, GCH)], sem).wait()
            sz = jnp.minimum(nt - off, SCH)

            def edge(i, carry3):
                l = lv[pl.ds(i, 16)][0]
                for j in range(FCH // 16):
                    sl = pl.ds(j * 16, 16)
                    v = rows[i, sl]
                    plsc.addupdate(st_s.at[l, sl], v)
                    plsc.addupdate(st_q.at[l, sl], v * v)
                    st_mn[l, sl] = jnp.minimum(st_mn[l, sl], v)
                    st_mx[l, sl] = jnp.maximum(st_mx[l, sl], v)
                return carry3
            lax.fori_loop(0, sz, edge, 0)
            return carry
        lax.fori_loop(0, nchunks, chunk, 0)

        row0 = pl.multiple_of(w * NPR, 8)
        col = pl.multiple_of(k * FCH, FCH)
        pltpu.sync_copy(st_s, out_hbm.at[0, pl.ds(row0, NPR), pl.ds(col, FCH)])
        pltpu.sync_copy(st_q, out_hbm.at[1, pl.ds(row0, NPR), pl.ds(col, FCH)])
        pltpu.sync_copy(st_mn, out_hbm.at[2, pl.ds(row0, NPR), pl.ds(col, FCH)])
        pltpu.sync_copy(st_mx, out_hbm.at[3, pl.ds(row0, NPR), pl.ds(col, FCH)])
        return carry0

    lax.fori_loop(0, NFP, fpass, 0)


def _tc_mm(h, wcat, bias, bn=None):
    blk = 1000
    grid = (N // blk,)

    def body_plain(h_ref, w_ref, b_ref, a_ref, bp_ref):
        ab = jnp.dot(h_ref[...], w_ref[...], preferred_element_type=jnp.float32)
        a_ref[...] = ab[:, :400] + b_ref[...]
        bp_ref[...] = ab[:, 400:]

    def body_bn(h_ref, w_ref, b_ref, st_ref, g_ref, bb_ref, a_ref, bp_ref,
                h_out_ref):
        mu = st_ref[0:1, :]
        var = st_ref[1:2, :]
        hx = (h_ref[...] - mu) * jax.lax.rsqrt(var + 1e-5)
        hx = jnp.maximum(hx * g_ref[...] + bb_ref[...], 0.0)
        h_out_ref[...] = hx
        ab = jnp.dot(hx, w_ref[...], preferred_element_type=jnp.float32)
        a_ref[...] = ab[:, :400] + b_ref[...]
        bp_ref[...] = ab[:, 400:]

    hspec = pl.BlockSpec((blk, F_IN), lambda i: (i, 0))
    wspec = pl.BlockSpec((F_IN, 848), lambda i: (0, 0))
    bspec = pl.BlockSpec((1, 400), lambda i: (0, 0))
    aspec = pl.BlockSpec((blk, 400), lambda i: (i, 0))
    bpspec = pl.BlockSpec((blk, 448), lambda i: (i, 0))
    if bn is None:
        return pl.pallas_call(
            body_plain, grid=grid, name="tc_mm_plain",
            in_specs=[hspec, wspec, bspec],
            out_specs=[aspec, bpspec],
            out_shape=[jax.ShapeDtypeStruct((N, 400), jnp.float32),
                       jax.ShapeDtypeStruct((N, 448), jnp.float32)],
        )(h, wcat, bias.reshape(1, 400))
    stats, g, bb = bn
    return pl.pallas_call(
        body_bn, grid=grid, name="tc_mm_bn",
        in_specs=[hspec, wspec, bspec,
                  pl.BlockSpec((2, F_IN), lambda i: (0, 0)),
                  pl.BlockSpec((1, F_IN), lambda i: (0, 0)),
                  pl.BlockSpec((1, F_IN), lambda i: (0, 0))],
        out_specs=[aspec, bpspec, pl.BlockSpec((blk, F_IN), lambda i: (i, 0))],
        out_shape=[jax.ShapeDtypeStruct((N, 400), jnp.float32),
                   jax.ShapeDtypeStruct((N, 448), jnp.float32),
                   jax.ShapeDtypeStruct((N, F_IN), jnp.float32)],
    )(h, wcat, bias.reshape(1, 400), stats, g.reshape(1, F_IN),
      bb.reshape(1, F_IN))


def _tc_post(a, ssum, ssq, smn, smx, cnt, x, p0r, wplain, wamp, watt,
             pb_flat, clw, clb):
    blk = 400
    grid = (N // blk,)

    def body(a_ref, s_ref, q_ref, mn_ref, mx_ref, c_ref, x_ref, p0_ref,
             wp_ref, wa_ref, wt_ref, pb_ref, clw_ref, clb_ref,
             y_ref, st_ref, acc):
        step = pl.program_id(0)
        av = a_ref[...]
        cnt_col = c_ref[...]
        deg = jnp.maximum(cnt_col, 1.0)
        has = cnt_col > 0.0
        sb = s_ref[...][:, :400]
        qb = q_ref[...][:, :400]
        mean = (cnt_col * av + sb) / deg
        meansq = (cnt_col * av * av + 2.0 * av * sb + qb) / deg
        std = jnp.sqrt(jnp.maximum(meansq - mean * mean, 0.0) + 1e-5)
        mn = jnp.where(has, av + mn_ref[...][:, :400], 0.0)
        mx = jnp.where(has, av + mx_ref[...][:, :400], 0.0)

        aggc = jnp.concatenate([mean, mn, mx, std], axis=1)

        logd = jnp.log(deg + 1.0)
        amp = logd / AVG_DEG_LOG
        att = AVG_DEG_LOG / logd

        y5 = (jnp.dot(x_ref[...], p0_ref[...],
                      preferred_element_type=jnp.float32)
              + jnp.dot(aggc, wp_ref[...], preferred_element_type=jnp.float32)
              + amp * jnp.dot(aggc, wa_ref[...],
                              preferred_element_type=jnp.float32)
              + att * jnp.dot(aggc, wt_ref[...],
                              preferred_element_type=jnp.float32)
              + pb_ref[...])
        y = jnp.dot(y5, clw_ref[...], preferred_element_type=jnp.float32) \
            + clb_ref[...]
        y_ref[...] = y

        @pl.when(step == 0)
        def _():
            acc[...] = jnp.zeros_like(acc)
        acc[0, :] += jnp.sum(y, axis=0)
        acc[1, :] += jnp.sum(y * y, axis=0)
        st_ref[...] = acc[...]

    return pl.pallas_call(
        body, grid=grid, name="tc_post",
        in_specs=[
            pl.BlockSpec((blk, 400), lambda i: (i, 0)),
            pl.BlockSpec((blk, 448), lambda i: (i, 0)),
            pl.BlockSpec((blk, 448), lambda i: (i, 0)),
            pl.BlockSpec((blk, 448), lambda i: (i, 0)),
            pl.BlockSpec((blk, 448), lambda i: (i, 0)),
            pl.BlockSpec((blk, 1), lambda i: (i, 0)),
            pl.BlockSpec((blk, F_IN), lambda i: (i, 0)),
            pl.BlockSpec((F_IN, 80), lambda i: (0, 0)),
            pl.BlockSpec((1600, 80), lambda i: (0, 0)),
            pl.BlockSpec((1600, 80), lambda i: (0, 0)),
            pl.BlockSpec((1600, 80), lambda i: (0, 0)),
            pl.BlockSpec((1, 80), lambda i: (0, 0)),
            pl.BlockSpec((80, 80), lambda i: (0, 0)),
            pl.BlockSpec((1, 80), lambda i: (0, 0)),
        ],
        out_specs=[pl.BlockSpec((blk, F_IN), lambda i: (i, 0)),
                   pl.BlockSpec((2, F_IN), lambda i: (0, 0))],
        out_shape=[jax.ShapeDtypeStruct((N, F_IN), jnp.float32),
                   jax.ShapeDtypeStruct((2, F_IN), jnp.float32)],
        scratch_shapes=[pltpu.VMEM((2, F_IN), jnp.float32)],
    )(a, ssum, ssq, smn, smx, cnt, x, p0r, wplain, wamp, watt,
      pb_flat.reshape(1, 80), clw, clb.reshape(1, 80))


NPOOL = 10240


def _tc_pool(y2, stats2, g2, b2, batch, lin_w, lin_b):
    blk = 640
    grid = (NPOOL // blk,)

    def body(y_ref, st_ref, g_ref, bb_ref, bt_ref, lw_ref, lb_ref, o_ref,
             pmax, pcnt):
        step = pl.program_id(0)

        @pl.when(step == 0)
        def _():
            pmax[...] = jnp.full_like(pmax, -BIGF)
            pcnt[...] = jnp.zeros_like(pcnt)

        mu = st_ref[0:1, :]
        var = st_ref[1:2, :]
        h = (y_ref[...] - mu) * jax.lax.rsqrt(var + 1e-5)
        h = jnp.maximum(h * g_ref[...] + bb_ref[...], 0.0)

        bt = bt_ref[...]
        for g in range(NUM_GRAPHS):
            maskg = bt == g
            cand = jnp.where(maskg, h, -BIGF)
            mg = jnp.max(cand, axis=0, keepdims=True)
            pmax[g:g + 1, :] = jnp.maximum(pmax[g:g + 1, :], mg)
            pcnt[g:g + 1, :] += jnp.sum(maskg.astype(jnp.float32), axis=0,
                                        keepdims=True)

        @pl.when(step == grid[0] - 1)
        def _():
            pooled = jnp.where(pcnt[...] > 0.0, pmax[...], 0.0)
            o_ref[...] = jnp.dot(pooled, lw_ref[...],
                                 preferred_element_type=jnp.float32) \
                + lb_ref[...]

    return pl.pallas_call(
        body, grid=grid, name="tc_pool",
        in_specs=[
            pl.BlockSpec((blk, F_IN), lambda i: (i, 0)),
            pl.BlockSpec((2, F_IN), lambda i: (0, 0)),
            pl.BlockSpec((1, F_IN), lambda i: (0, 0)),
            pl.BlockSpec((1, F_IN), lambda i: (0, 0)),
            pl.BlockSpec((blk, 1), lambda i: (i, 0)),
            pl.BlockSpec((F_IN, 20), lambda i: (0, 0)),
            pl.BlockSpec((1, 20), lambda i: (0, 0)),
        ],
        out_specs=pl.BlockSpec((NUM_GRAPHS, 20), lambda i: (0, 0)),
        out_shape=jax.ShapeDtypeStruct((NUM_GRAPHS, 20), jnp.float32),
        scratch_shapes=[pltpu.VMEM((NUM_GRAPHS, F_IN), jnp.float32),
                        pltpu.VMEM((NUM_GRAPHS, 1), jnp.float32)],
    )(y2, stats2, g2.reshape(1, F_IN), b2.reshape(1, F_IN), batch,
      lin_w, lin_b.reshape(1, 20))


def _prep_pre(pre_w, pre_b):
    wd = jnp.transpose(pre_w[:, :F_IN, :], (1, 0, 2)).reshape(F_IN, 400)
    ws = jnp.transpose(pre_w[:, F_IN:, :], (1, 0, 2)).reshape(F_IN, 400)
    ws = jnp.pad(ws, ((0, 0), (0, 48)))
    wcat = jnp.concatenate([wd, ws], axis=1)
    return wcat, pre_b.reshape(400)


def _prep_post(post_w):
    p0r = jnp.transpose(post_w[:, :80, :], (1, 0, 2)).reshape(80, 80)
    eye = jnp.eye(TOWERS, dtype=jnp.float32)

    def bd80(sl):
        return (sl[:, :, None, :] * eye[:, None, :, None]).reshape(400, 80)

    def group(base):
        return jnp.concatenate(
            [bd80(post_w[:, base + 80 * a: base + 80 * (a + 1), :])
             for a in range(4)], axis=0)
    wplain = group(80)
    wamp = group(400)
    watt = group(720)
    return p0r, wplain, wamp, watt


def _layer(h_act_or_y, wcat, bias, post_parts, post_b, clw, clb,
           srcs, ldsts, tcnt, cnt_col, bn=None):
    outs = _tc_mm(h_act_or_y, wcat, bias, bn=bn)
    if bn is None:
        a, bp = outs
        x_act = h_act_or_y
    else:
        a, bp, x_act = outs
    b7 = bp.reshape(N, NFP, FCH).transpose(1, 0, 2).reshape(NFP * N, FCH)
    segred = _sc_segred(b7, srcs, ldsts, tcnt)
    segred = segred.reshape(4, NW, NPR, 448)[:, :, :NP, :].reshape(4, NPAD, 448)
    ssum, ssq, smn, smx = (segred[i, :N, :] for i in range(4))
    p0r, wplain, wamp, watt = post_parts
    return _tc_post(a, ssum, ssq, smn, smx, cnt_col, x_act, p0r, wplain,
                    wamp, watt, post_b.reshape(80), clw, clb)


def kernel(x, edge_index, batch, node_emb, pre_w1, pre_b1, post_w1, post_b1,
           conv_lin_w1, conv_lin_b1, bn_g1, bn_b1,
           pre_w2, pre_b2, post_w2, post_b2, conv_lin_w2, conv_lin_b2,
           bn_g2, bn_b2, lin_w, lin_b):
    table16 = jnp.pad(node_emb, ((0, 0), (0, 14)))
    emb16 = _sc_emb(table16, x.reshape(-1))
    h1 = emb16[:, :2].reshape(N, F_IN)

    srcs, ldsts, cntp, tcnt = _sc_csr(edge_index)
    cnt_col = cntp[:N].reshape(N, 1)

    wcat1, bias1 = _prep_pre(pre_w1, pre_b1)
    wcat2, bias2 = _prep_pre(pre_w2, pre_b2)
    parts1 = _prep_post(post_w1)
    parts2 = _prep_post(post_w2)

    y1, stats1 = _layer(h1, wcat1, bias1, parts1, post_b1, conv_lin_w1,
                        conv_lin_b1, srcs, ldsts, tcnt, cnt_col)
    y2, stats2 = _layer(y1, wcat2, bias2, parts2, post_b2, conv_lin_w2,
                        conv_lin_b2, srcs, ldsts, tcnt, cnt_col,
                        bn=(_fix_stats(stats1), bn_g1, bn_b1))
    y2p = jnp.pad(y2, ((0, NPOOL - N), (0, 0)))
    batchp = jnp.pad(batch.reshape(N, 1), ((0, NPOOL - N), (0, 0)),
                     constant_values=127)
    out = _tc_pool(y2p, _fix_stats(stats2), bn_g2, bn_b2, batchp,
                   lin_w, lin_b)
    return out


def _fix_stats(stats_raw):
    mu = stats_raw[0] / float(N)
    var = stats_raw[1] / float(N) - mu * mu
    return jnp.stack([mu, var])


# speedup vs baseline: 31.5223x; 1.2102x over previous
import functools
import numpy as np
import jax
import jax.numpy as jnp
from jax import lax
from jax.experimental import pallas as pl
from jax.experimental.pallas import tpu as pltpu
from jax.experimental.pallas import tpu_sc as plsc

N = 10000
E = 160000
TOWERS = 5
F_IN = 80
F_OUT = 16
NUM_GRAPHS = 64
VOCAB = 10000
AVG_DEG_LOG = float(np.log(17.0))

NW = 32
NP = 320
NPR = NP + 8
NPAD = NW * NP
FCH = 64
NFP = 7
FP = NFP * FCH
ECH = 2000
LCAP = E + 4096
GCH = 128
SCH = 512

@functools.lru_cache(maxsize=None)
def _mesh():
    return plsc.VectorSubcoreMesh(core_axis_name="c", subcore_axis_name="s")


def _wid():
    return lax.axis_index("s") * 2 + lax.axis_index("c")


NIDX = N * 40
NGCH = NIDX // GCH


@functools.lru_cache(maxsize=None)
def _emb_kernel():
    @functools.partial(
        pl.kernel, mesh=_mesh(),
        compiler_params=pltpu.CompilerParams(use_tc_tiling_on_sc=False, needs_layout_passes=False),
        name="sc_emb",
        out_type=jax.ShapeDtypeStruct((NIDX, 16), jnp.float32),
        scratch_types=[
            pltpu.VMEM((GCH,), jnp.int32),
            pltpu.VMEM((GCH, 16), jnp.float32),
            pltpu.SemaphoreType.DMA,
        ],
    )
    def k(table_hbm, idx_hbm, out_hbm, idx_v, rows_v, sem):
        w = _wid()
        nci = (NGCH - w + NW - 1) // NW

        def body(c, carry):
            ci = w + c * NW
            base = pl.multiple_of(ci * GCH, GCH)
            pltpu.sync_copy(idx_hbm.at[pl.ds(base, GCH)], idx_v)
            pltpu.async_copy(table_hbm.at[idx_v], rows_v, sem).wait()
            pltpu.sync_copy(rows_v, out_hbm.at[pl.ds(base, GCH)])
            return carry

        lax.fori_loop(0, nci, body, 0)
    return k


def _sc_emb(table16, idx):
    return _emb_kernel()(table16, idx)


@functools.lru_cache(maxsize=None)
def _csr_kernel():
    @functools.partial(
        pl.kernel, mesh=_mesh(),
        compiler_params=pltpu.CompilerParams(use_tc_tiling_on_sc=False, needs_layout_passes=False),
        name="sc_csr",
        out_type=[
            jax.ShapeDtypeStruct((NW, LCAP), jnp.int32),
            jax.ShapeDtypeStruct((NW, LCAP), jnp.int32),
            jax.ShapeDtypeStruct((NPAD,), jnp.float32),
            jax.ShapeDtypeStruct((NW * 8,), jnp.int32),
        ],
        scratch_types=[
            pltpu.VMEM((ECH,), jnp.int32),
            pltpu.VMEM((ECH,), jnp.int32),
            pltpu.VMEM((ECH + 16,), jnp.int32),
            pltpu.VMEM((ECH + 16,), jnp.int32),
            pltpu.VMEM((NP + 16,), jnp.float32),
            pltpu.VMEM((16,), jnp.int32),
        ],
    )
    def k(edge_hbm, srcs_hbm, ldst_hbm, cnt_hbm, tcnt_hbm,
          sv, dv, csv, clv, cntv, tmpv):
        _csr_body(edge_hbm, srcs_hbm, ldst_hbm, cnt_hbm, tcnt_hbm,
                  sv, dv, csv, clv, cntv, tmpv)
    return k


def _sc_csr(edge_index):
    return _csr_kernel()(edge_index)


def _csr_body(edge_hbm, srcs_hbm, ldst_hbm, cnt_hbm, tcnt_hbm,
              sv, dv, csv, clv, cntv, tmpv):
    w = _wid()
    lo = pl.multiple_of(w * NP, NP)

    def zbody(i, carry):
        cntv[pl.ds(i * 16, 16)] = jnp.zeros((16,), jnp.float32)
        return carry
    lax.fori_loop(0, (NP + 16) // 16, zbody, 0)

    lanes = lax.iota(jnp.int32, 16)
    filler_ld = NP + (lanes % 8)

    def chunk(c, w_off):
        pltpu.sync_copy(edge_hbm.at[0, pl.ds(c * ECH, ECH)], sv)
        pltpu.sync_copy(edge_hbm.at[1, pl.ds(c * ECH, ECH)], dv)

        def step(j, cur):
            d = dv[pl.ds(j * 16, 16)]
            s = sv[pl.ds(j * 16, 16)]
            m = (d >= lo) & (d < lo + NP)
            ld = jnp.where(m, d - lo, NP)
            rank = plsc.cumsum(jnp.where(m, 1, 0).astype(jnp.int32))
            pos = jnp.where(m, cur + rank - 1, cur + 15)
            plsc.store_scatter(csv, [pos], s)
            plsc.store_scatter(clv, [pos], ld)
            plsc.addupdate_scatter(cntv, [ld], jnp.ones((16,), jnp.float32))
            return cur + rank[15]

        cur = lax.fori_loop(0, ECH // 16, step, 0)
        csv[pl.ds(cur, 16)] = jnp.full((16,), w, jnp.int32)
        clv[pl.ds(cur, 16)] = filler_ld
        cur_pad = ((cur + 15) // 16) * 16
        w_off = pl.multiple_of(w_off, 16)
        pltpu.sync_copy(csv, srcs_hbm.at[w, pl.ds(w_off, ECH + 16)])
        pltpu.sync_copy(clv, ldst_hbm.at[w, pl.ds(w_off, ECH + 16)])
        return w_off + cur_pad

    total = lax.fori_loop(0, E // ECH, chunk, 0)
    tmpv[...] = jnp.broadcast_to(total, (16,)).astype(jnp.int32)
    pltpu.sync_copy(tmpv.at[pl.ds(0, 8)],
                    tcnt_hbm.at[pl.ds(pl.multiple_of(w * 8, 8), 8)])
    pltpu.sync_copy(cntv.at[pl.ds(0, NP)], cnt_hbm.at[pl.ds(lo, NP)])


BIGF = 3.0e38


@functools.lru_cache(maxsize=None)
def _segred_kernel():
    @functools.partial(
        pl.kernel, mesh=_mesh(),
        compiler_params=pltpu.CompilerParams(use_tc_tiling_on_sc=False, needs_layout_passes=False),
        name="sc_segred",
        out_type=jax.ShapeDtypeStruct((4, NW * NPR, FP), jnp.float32),
        scratch_types=[
            pltpu.VMEM((SCH,), jnp.int32),
            pltpu.VMEM((SCH + 16,), jnp.int32),
            pltpu.VMEM((4, GCH), jnp.int32),
            pltpu.VMEM((SCH, FCH), jnp.float32),
            pltpu.VMEM((NPR, FCH), jnp.float32),
            pltpu.VMEM((NPR, FCH), jnp.float32),
            pltpu.VMEM((NPR, FCH), jnp.float32),
            pltpu.VMEM((NPR, FCH), jnp.float32),
            pltpu.VMEM((16,), jnp.int32),
            pltpu.SemaphoreType.DMA,
        ],
    )
    def k(b7_hbm, srcs_hbm, ldst_hbm, tcnt_hbm, out_hbm,
          sv, lv, idxv, rows, st_s, st_q, st_mn, st_mx, ntv, sem):
        _segred_body(b7_hbm, srcs_hbm, ldst_hbm, tcnt_hbm, out_hbm,
                     sv, lv, idxv, rows, st_s, st_q, st_mn, st_mx, ntv, sem)
    return k


def _sc_segred(b7, srcs, ldsts, tcnt):
    return _segred_kernel()(b7, srcs, ldsts, tcnt)


def _segred_body(b7_hbm, srcs_hbm, ldst_hbm, tcnt_hbm, out_hbm,
                 sv, lv, idxv, rows, st_s, st_q, st_mn, st_mx, ntv, sem):
    w = _wid()
    pltpu.sync_copy(tcnt_hbm.at[pl.ds(pl.multiple_of(w * 8, 8), 8)],
                    ntv.at[pl.ds(0, 8)])
    nt = ntv[pl.ds(0, 16)][0]
    nchunks = (nt + SCH - 1) // SCH
    lanes = lax.iota(jnp.int32, 16)

    def fpass(k, carry0):
        def zrow(i, carry):
            for j in range(FCH // 16):
                sl = pl.ds(j * 16, 16)
                st_s[i, sl] = jnp.zeros((16,), jnp.float32)
                st_q[i, sl] = jnp.zeros((16,), jnp.float32)
                st_mn[i, sl] = jnp.full((16,), BIGF, jnp.float32)
                st_mx[i, sl] = jnp.full((16,), -BIGF, jnp.float32)
            return carry
        lax.fori_loop(0, NPR, zrow, 0)

        kbase = k * N

        def chunk(c, carry):
            off = pl.multiple_of(c * SCH, SCH)
            pltpu.sync_copy(srcs_hbm.at[w, pl.ds(off, SCH)], sv)
            pltpu.sync_copy(ldst_hbm.at[w, pl.ds(off, SCH)],
                            lv.at[pl.ds(0, SCH)])
            def mkidx(j, carry2):
                pos = off + j * 16 + lanes
                valid = pos < nt
                s = jnp.where(valid, sv[pl.ds(j * 16, 16)], w)
                idxv[j // (GCH // 16), pl.ds((j % (GCH // 16)) * 16, 16)] = (
                    s + kbase)
                lvec = lv[pl.ds(j * 16, 16)]
                lv[pl.ds(j * 16, 16)] = jnp.where(valid, lvec, NP)
                return carry2
            lax.fori_loop(0, SCH // 16, mkidx, 0)
            descs = [pltpu.async_copy(b7_hbm.at[idxv.at[q]],
                                      rows.at[pl.ds(q * GCH, GCH)], sem)
                     for q in range(SCH // GCH)]
            for d in descs:
                d.wait()

            def group(g, carry3):
                base16 = pl.multiple_of(g * 16, 16)
                lvec = lv[pl.ds(base16, 16)]
                for j16 in range(16):
                    l = lvec[j16]
                    i = base16 + j16
                    for j in range(FCH // 16):
                        sl = pl.ds(j * 16, 16)
                        v = rows[i, sl]
                        plsc.addupdate(st_s.at[l, sl], v)
                        plsc.addupdate(st_q.at[l, sl], v * v)
                        st_mn[l, sl] = jnp.minimum(st_mn[l, sl], v)
                        st_mx[l, sl] = jnp.maximum(st_mx[l, sl], v)
                return carry3
            lax.fori_loop(0, SCH // 16, group, 0)
            return carry
        lax.fori_loop(0, nchunks, chunk, 0)

        row0 = pl.multiple_of(w * NPR, 8)
        col = pl.multiple_of(k * FCH, FCH)
        pltpu.sync_copy(st_s, out_hbm.at[0, pl.ds(row0, NPR), pl.ds(col, FCH)])
        pltpu.sync_copy(st_q, out_hbm.at[1, pl.ds(row0, NPR), pl.ds(col, FCH)])
        pltpu.sync_copy(st_mn, out_hbm.at[2, pl.ds(row0, NPR), pl.ds(col, FCH)])
        pltpu.sync_copy(st_mx, out_hbm.at[3, pl.ds(row0, NPR), pl.ds(col, FCH)])
        return carry0

    lax.fori_loop(0, NFP, fpass, 0)


def _tc_mm(h, wcat, bias, bn=None):
    blk = 1000
    grid = (N // blk,)

    def body_plain(h_ref, w_ref, b_ref, a_ref, bp_ref):
        ab = jnp.dot(h_ref[...], w_ref[...], preferred_element_type=jnp.float32)
        a_ref[...] = ab[:, :400] + b_ref[...]
        bp_ref[...] = ab[:, 400:]

    def body_bn(h_ref, w_ref, b_ref, st_ref, g_ref, bb_ref, a_ref, bp_ref,
                h_out_ref):
        mu = st_ref[0:1, :]
        var = st_ref[1:2, :]
        hx = (h_ref[...] - mu) * jax.lax.rsqrt(var + 1e-5)
        hx = jnp.maximum(hx * g_ref[...] + bb_ref[...], 0.0)
        h_out_ref[...] = hx
        ab = jnp.dot(hx, w_ref[...], preferred_element_type=jnp.float32)
        a_ref[...] = ab[:, :400] + b_ref[...]
        bp_ref[...] = ab[:, 400:]

    hspec = pl.BlockSpec((blk, F_IN), lambda i: (i, 0))
    wspec = pl.BlockSpec((F_IN, 848), lambda i: (0, 0))
    bspec = pl.BlockSpec((1, 400), lambda i: (0, 0))
    aspec = pl.BlockSpec((blk, 400), lambda i: (i, 0))
    bpspec = pl.BlockSpec((blk, 448), lambda i: (i, 0))
    if bn is None:
        return pl.pallas_call(
            body_plain, grid=grid, name="tc_mm_plain",
            in_specs=[hspec, wspec, bspec],
            out_specs=[aspec, bpspec],
            out_shape=[jax.ShapeDtypeStruct((N, 400), jnp.float32),
                       jax.ShapeDtypeStruct((N, 448), jnp.float32)],
        )(h, wcat, bias.reshape(1, 400))
    stats, g, bb = bn
    return pl.pallas_call(
        body_bn, grid=grid, name="tc_mm_bn",
        in_specs=[hspec, wspec, bspec,
                  pl.BlockSpec((2, F_IN), lambda i: (0, 0)),
                  pl.BlockSpec((1, F_IN), lambda i: (0, 0)),
                  pl.BlockSpec((1, F_IN), lambda i: (0, 0))],
        out_specs=[aspec, bpspec, pl.BlockSpec((blk, F_IN), lambda i: (i, 0))],
        out_shape=[jax.ShapeDtypeStruct((N, 400), jnp.float32),
                   jax.ShapeDtypeStruct((N, 448), jnp.float32),
                   jax.ShapeDtypeStruct((N, F_IN), jnp.float32)],
    )(h, wcat, bias.reshape(1, 400), stats, g.reshape(1, F_IN),
      bb.reshape(1, F_IN))


def _tc_post(a, ssum, ssq, smn, smx, cnt, x, p0r, wplain, wamp, watt,
             pb_flat, clw, clb):
    blk = 400
    grid = (N // blk,)

    def body(a_ref, s_ref, q_ref, mn_ref, mx_ref, c_ref, x_ref, p0_ref,
             wp_ref, wa_ref, wt_ref, pb_ref, clw_ref, clb_ref,
             y_ref, st_ref, acc):
        step = pl.program_id(0)
        av = a_ref[...]
        cnt_col = c_ref[...]
        deg = jnp.maximum(cnt_col, 1.0)
        has = cnt_col > 0.0
        sb = s_ref[...][:, :400]
        qb = q_ref[...][:, :400]
        mean = (cnt_col * av + sb) / deg
        meansq = (cnt_col * av * av + 2.0 * av * sb + qb) / deg
        std = jnp.sqrt(jnp.maximum(meansq - mean * mean, 0.0) + 1e-5)
        mn = jnp.where(has, av + mn_ref[...][:, :400], 0.0)
        mx = jnp.where(has, av + mx_ref[...][:, :400], 0.0)

        aggc = jnp.concatenate([mean, mn, mx, std], axis=1)

        logd = jnp.log(deg + 1.0)
        amp = logd / AVG_DEG_LOG
        att = AVG_DEG_LOG / logd

        y5 = (jnp.dot(x_ref[...], p0_ref[...],
                      preferred_element_type=jnp.float32)
              + jnp.dot(aggc, wp_ref[...], preferred_element_type=jnp.float32)
              + amp * jnp.dot(aggc, wa_ref[...],
                              preferred_element_type=jnp.float32)
              + att * jnp.dot(aggc, wt_ref[...],
                              preferred_element_type=jnp.float32)
              + pb_ref[...])
        y = jnp.dot(y5, clw_ref[...], preferred_element_type=jnp.float32) \
            + clb_ref[...]
        y_ref[...] = y

        @pl.when(step == 0)
        def _():
            acc[...] = jnp.zeros_like(acc)
        acc[0, :] += jnp.sum(y, axis=0)
        acc[1, :] += jnp.sum(y * y, axis=0)
        st_ref[...] = acc[...]

    return pl.pallas_call(
        body, grid=grid, name="tc_post",
        in_specs=[
            pl.BlockSpec((blk, 400), lambda i: (i, 0)),
            pl.BlockSpec((blk, 448), lambda i: (i, 0)),
            pl.BlockSpec((blk, 448), lambda i: (i, 0)),
            pl.BlockSpec((blk, 448), lambda i: (i, 0)),
            pl.BlockSpec((blk, 448), lambda i: (i, 0)),
            pl.BlockSpec((blk, 1), lambda i: (i, 0)),
            pl.BlockSpec((blk, F_IN), lambda i: (i, 0)),
            pl.BlockSpec((F_IN, 80), lambda i: (0, 0)),
            pl.BlockSpec((1600, 80), lambda i: (0, 0)),
            pl.BlockSpec((1600, 80), lambda i: (0, 0)),
            pl.BlockSpec((1600, 80), lambda i: (0, 0)),
            pl.BlockSpec((1, 80), lambda i: (0, 0)),
            pl.BlockSpec((80, 80), lambda i: (0, 0)),
            pl.BlockSpec((1, 80), lambda i: (0, 0)),
        ],
        out_specs=[pl.BlockSpec((blk, F_IN), lambda i: (i, 0)),
                   pl.BlockSpec((2, F_IN), lambda i: (0, 0))],
        out_shape=[jax.ShapeDtypeStruct((N, F_IN), jnp.float32),
                   jax.ShapeDtypeStruct((2, F_IN), jnp.float32)],
        scratch_shapes=[pltpu.VMEM((2, F_IN), jnp.float32)],
    )(a, ssum, ssq, smn, smx, cnt, x, p0r, wplain, wamp, watt,
      pb_flat.reshape(1, 80), clw, clb.reshape(1, 80))


NPOOL = 10240


def _tc_pool(y2, stats2, g2, b2, batch, lin_w, lin_b):
    blk = 640
    grid = (NPOOL // blk,)

    def body(y_ref, st_ref, g_ref, bb_ref, bt_ref, lw_ref, lb_ref, o_ref,
             pmax, pcnt):
        step = pl.program_id(0)

        @pl.when(step == 0)
        def _():
            pmax[...] = jnp.full_like(pmax, -BIGF)
            pcnt[...] = jnp.zeros_like(pcnt)

        mu = st_ref[0:1, :]
        var = st_ref[1:2, :]
        h = (y_ref[...] - mu) * jax.lax.rsqrt(var + 1e-5)
        h = jnp.maximum(h * g_ref[...] + bb_ref[...], 0.0)

        bt = bt_ref[...]
        for g in range(NUM_GRAPHS):
            maskg = bt == g
            cand = jnp.where(maskg, h, -BIGF)
            mg = jnp.max(cand, axis=0, keepdims=True)
            pmax[g:g + 1, :] = jnp.maximum(pmax[g:g + 1, :], mg)
            pcnt[g:g + 1, :] += jnp.sum(maskg.astype(jnp.float32), axis=0,
                                        keepdims=True)

        @pl.when(step == grid[0] - 1)
        def _():
            pooled = jnp.where(pcnt[...] > 0.0, pmax[...], 0.0)
            o_ref[...] = jnp.dot(pooled, lw_ref[...],
                                 preferred_element_type=jnp.float32) \
                + lb_ref[...]

    return pl.pallas_call(
        body, grid=grid, name="tc_pool",
        in_specs=[
            pl.BlockSpec((blk, F_IN), lambda i: (i, 0)),
            pl.BlockSpec((2, F_IN), lambda i: (0, 0)),
            pl.BlockSpec((1, F_IN), lambda i: (0, 0)),
            pl.BlockSpec((1, F_IN), lambda i: (0, 0)),
            pl.BlockSpec((blk, 1), lambda i: (i, 0)),
            pl.BlockSpec((F_IN, 20), lambda i: (0, 0)),
            pl.BlockSpec((1, 20), lambda i: (0, 0)),
        ],
        out_specs=pl.BlockSpec((NUM_GRAPHS, 20), lambda i: (0, 0)),
        out_shape=jax.ShapeDtypeStruct((NUM_GRAPHS, 20), jnp.float32),
        scratch_shapes=[pltpu.VMEM((NUM_GRAPHS, F_IN), jnp.float32),
                        pltpu.VMEM((NUM_GRAPHS, 1), jnp.float32)],
    )(y2, stats2, g2.reshape(1, F_IN), b2.reshape(1, F_IN), batch,
      lin_w, lin_b.reshape(1, 20))


def _prep_pre(pre_w, pre_b):
    wd = jnp.transpose(pre_w[:, :F_IN, :], (1, 0, 2)).reshape(F_IN, 400)
    ws = jnp.transpose(pre_w[:, F_IN:, :], (1, 0, 2)).reshape(F_IN, 400)
    ws = jnp.pad(ws, ((0, 0), (0, 48)))
    wcat = jnp.concatenate([wd, ws], axis=1)
    return wcat, pre_b.reshape(400)


def _prep_post(post_w):
    p0r = jnp.transpose(post_w[:, :80, :], (1, 0, 2)).reshape(80, 80)
    eye = jnp.eye(TOWERS, dtype=jnp.float32)

    def bd80(sl):
        return (sl[:, :, None, :] * eye[:, None, :, None]).reshape(400, 80)

    def group(base):
        return jnp.concatenate(
            [bd80(post_w[:, base + 80 * a: base + 80 * (a + 1), :])
             for a in range(4)], axis=0)
    wplain = group(80)
    wamp = group(400)
    watt = group(720)
    return p0r, wplain, wamp, watt


def _layer(h_act_or_y, wcat, bias, post_parts, post_b, clw, clb,
           srcs, ldsts, tcnt, cnt_col, bn=None):
    outs = _tc_mm(h_act_or_y, wcat, bias, bn=bn)
    if bn is None:
        a, bp = outs
        x_act = h_act_or_y
    else:
        a, bp, x_act = outs
    b7 = bp.reshape(N, NFP, FCH).transpose(1, 0, 2).reshape(NFP * N, FCH)
    segred = _sc_segred(b7, srcs, ldsts, tcnt)
    segred = segred.reshape(4, NW, NPR, 448)[:, :, :NP, :].reshape(4, NPAD, 448)
    ssum, ssq, smn, smx = (segred[i, :N, :] for i in range(4))
    p0r, wplain, wamp, watt = post_parts
    return _tc_post(a, ssum, ssq, smn, smx, cnt_col, x_act, p0r, wplain,
                    wamp, watt, post_b.reshape(80), clw, clb)


def kernel(x, edge_index, batch, node_emb, pre_w1, pre_b1, post_w1, post_b1,
           conv_lin_w1, conv_lin_b1, bn_g1, bn_b1,
           pre_w2, pre_b2, post_w2, post_b2, conv_lin_w2, conv_lin_b2,
           bn_g2, bn_b2, lin_w, lin_b):
    table16 = jnp.pad(node_emb, ((0, 0), (0, 14)))
    emb16 = _sc_emb(table16, x.reshape(-1))
    h1 = emb16[:, :2].reshape(N, F_IN)

    srcs, ldsts, cntp, tcnt = _sc_csr(edge_index)
    cnt_col = cntp[:N].reshape(N, 1)

    wcat1, bias1 = _prep_pre(pre_w1, pre_b1)
    wcat2, bias2 = _prep_pre(pre_w2, pre_b2)
    parts1 = _prep_post(post_w1)
    parts2 = _prep_post(post_w2)

    y1, stats1 = _layer(h1, wcat1, bias1, parts1, post_b1, conv_lin_w1,
                        conv_lin_b1, srcs, ldsts, tcnt, cnt_col)
    y2, stats2 = _layer(y1, wcat2, bias2, parts2, post_b2, conv_lin_w2,
                        conv_lin_b2, srcs, ldsts, tcnt, cnt_col,
                        bn=(_fix_stats(stats1), bn_g1, bn_b1))
    y2p = jnp.pad(y2, ((0, NPOOL - N), (0, 0)))
    batchp = jnp.pad(batch.reshape(N, 1), ((0, NPOOL - N), (0, 0)),
                     constant_values=127)
    out = _tc_pool(y2p, _fix_stats(stats2), bn_g2, bn_b2, batchp,
                   lin_w, lin_b)
    return out


def _fix_stats(stats_raw):
    mu = stats_raw[0] / float(N)
    var = stats_raw[1] / float(N) - mu * mu
    return jnp.stack([mu, var])


# speedup vs baseline: 36.8579x; 1.1693x over previous
import functools
import numpy as np
import jax
import jax.numpy as jnp
from jax import lax
from jax.experimental import pallas as pl
from jax.experimental.pallas import tpu as pltpu
from jax.experimental.pallas import tpu_sc as plsc

N = 10000
E = 160000
TOWERS = 5
F_IN = 80
F_OUT = 16
NUM_GRAPHS = 64
VOCAB = 10000
AVG_DEG_LOG = float(np.log(17.0))

NW = 32
NP = 320
NPR = NP + 8
NPAD = NW * NP
FCH = 64
NFP = 7
FP = NFP * FCH
ECH = 4000
LCAP = E + 8192
GCH = 128
SCH = 256

@functools.lru_cache(maxsize=None)
def _mesh():
    return plsc.VectorSubcoreMesh(core_axis_name="c", subcore_axis_name="s")


def _wid():
    return lax.axis_index("s") * 2 + lax.axis_index("c")


NIDX = N * 40
NGCH = NIDX // GCH


@functools.lru_cache(maxsize=None)
def _emb_kernel():
    @functools.partial(
        pl.kernel, mesh=_mesh(),
        compiler_params=pltpu.CompilerParams(use_tc_tiling_on_sc=False, needs_layout_passes=False),
        name="sc_emb",
        out_type=jax.ShapeDtypeStruct((NIDX, 16), jnp.float32),
        scratch_types=[
            pltpu.VMEM((GCH,), jnp.int32),
            pltpu.VMEM((GCH, 16), jnp.float32),
            pltpu.SemaphoreType.DMA,
        ],
    )
    def k(table_hbm, idx_hbm, out_hbm, idx_v, rows_v, sem):
        w = _wid()
        nci = (NGCH - w + NW - 1) // NW

        def body(c, carry):
            ci = w + c * NW
            base = pl.multiple_of(ci * GCH, GCH)
            pltpu.sync_copy(idx_hbm.at[pl.ds(base, GCH)], idx_v)
            pltpu.async_copy(table_hbm.at[idx_v], rows_v, sem).wait()
            pltpu.sync_copy(rows_v, out_hbm.at[pl.ds(base, GCH)])
            return carry

        lax.fori_loop(0, nci, body, 0)
    return k


def _sc_emb(table16, idx):
    return _emb_kernel()(table16, idx)


@functools.lru_cache(maxsize=None)
def _csr_kernel():
    @functools.partial(
        pl.kernel, mesh=_mesh(),
        compiler_params=pltpu.CompilerParams(use_tc_tiling_on_sc=False, needs_layout_passes=False),
        name="sc_csr",
        out_type=[
            jax.ShapeDtypeStruct((NW, LCAP), jnp.int32),
            jax.ShapeDtypeStruct((NW, LCAP), jnp.int32),
            jax.ShapeDtypeStruct((NPAD,), jnp.float32),
            jax.ShapeDtypeStruct((NW * 8,), jnp.int32),
        ],
        scratch_types=[
            pltpu.VMEM((2, ECH), jnp.int32),
            pltpu.VMEM((2, ECH), jnp.int32),
            pltpu.VMEM((ECH + 16,), jnp.int32),
            pltpu.VMEM((ECH + 16,), jnp.int32),
            pltpu.VMEM((NP + 16,), jnp.float32),
            pltpu.VMEM((16,), jnp.int32),
            pltpu.SemaphoreType.DMA,
            pltpu.SemaphoreType.DMA,
        ],
    )
    def k(edge_hbm, srcs_hbm, ldst_hbm, cnt_hbm, tcnt_hbm,
          sv, dv, csv, clv, cntv, tmpv, sem0, sem1):
        _csr_body(edge_hbm, srcs_hbm, ldst_hbm, cnt_hbm, tcnt_hbm,
                  sv, dv, csv, clv, cntv, tmpv, (sem0, sem1))
    return k


def _sc_csr(edge_index):
    return _csr_kernel()(edge_index)


def _csr_body(edge_hbm, srcs_hbm, ldst_hbm, cnt_hbm, tcnt_hbm,
              sv, dv, csv, clv, cntv, tmpv, sems):
    w = _wid()
    lo = pl.multiple_of(w * NP, NP)

    def zbody(i, carry):
        cntv[pl.ds(i * 16, 16)] = jnp.zeros((16,), jnp.float32)
        return carry
    lax.fori_loop(0, (NP + 16) // 16, zbody, 0)

    lanes = lax.iota(jnp.int32, 16)
    filler_ld = NP + (lanes % 8)

    def fire(c, b):
        off = pl.multiple_of(jnp.minimum(c * ECH, E - ECH), 16)
        pltpu.async_copy(edge_hbm.at[0, pl.ds(off, ECH)], sv.at[b], sems[b])
        pltpu.async_copy(edge_hbm.at[1, pl.ds(off, ECH)], dv.at[b], sems[b])

    def drain(b):
        pltpu.make_async_copy(edge_hbm.at[0, pl.ds(0, ECH)], sv.at[b],
                              sems[b]).wait()
        pltpu.make_async_copy(edge_hbm.at[1, pl.ds(0, ECH)], dv.at[b],
                              sems[b]).wait()

    def process(b, w_off):
        def step(j, cur):
            d = dv[b, pl.ds(j * 16, 16)]
            s = sv[b, pl.ds(j * 16, 16)]
            m = (d >= lo) & (d < lo + NP)
            ld = jnp.where(m, d - lo, NP)
            rank = plsc.cumsum(jnp.where(m, 1, 0).astype(jnp.int32))
            pos = jnp.where(m, cur + rank - 1, cur + 15)
            plsc.store_scatter(csv, [pos], s)
            plsc.store_scatter(clv, [pos], ld)
            plsc.addupdate_scatter(cntv, [ld], jnp.ones((16,), jnp.float32))
            return cur + rank[15]

        cur = lax.fori_loop(0, ECH // 16, step, 0)
        csv[pl.ds(cur, 16)] = jnp.full((16,), w, jnp.int32)
        clv[pl.ds(cur, 16)] = filler_ld
        cur_pad = ((cur + 15) // 16) * 16
        w_off = pl.multiple_of(w_off, 16)
        pltpu.sync_copy(csv, srcs_hbm.at[w, pl.ds(w_off, ECH + 16)])
        pltpu.sync_copy(clv, ldst_hbm.at[w, pl.ds(w_off, ECH + 16)])
        return w_off + cur_pad

    fire(0, 0)

    def outer(t, w_off):
        for b in (0, 1):
            c = 2 * t + b
            fire(c + 1, (b + 1) % 2)
            drain(b)
            w_off = process(b, w_off)
        return w_off

    total = lax.fori_loop(0, (E // ECH) // 2, outer, 0)
    drain(0)
    tmpv[...] = jnp.broadcast_to(total, (16,)).astype(jnp.int32)
    pltpu.sync_copy(tmpv.at[pl.ds(0, 8)],
                    tcnt_hbm.at[pl.ds(pl.multiple_of(w * 8, 8), 8)])
    pltpu.sync_copy(cntv.at[pl.ds(0, NP)], cnt_hbm.at[pl.ds(lo, NP)])


BIGF = 3.0e38


@functools.lru_cache(maxsize=None)
def _segred_kernel():
    @functools.partial(
        pl.kernel, mesh=_mesh(),
        compiler_params=pltpu.CompilerParams(use_tc_tiling_on_sc=False, needs_layout_passes=False),
        name="sc_segred",
        out_type=jax.ShapeDtypeStruct((4, NW * NPR, FP), jnp.float32),
        scratch_types=[
            pltpu.VMEM((2, SCH), jnp.int32),
            pltpu.VMEM((2, SCH + 16), jnp.int32),
            pltpu.VMEM((2, SCH // GCH, GCH), jnp.int32),
            pltpu.VMEM((2, SCH, FCH), jnp.float32),
            pltpu.VMEM((NPR, FCH), jnp.float32),
            pltpu.VMEM((NPR, FCH), jnp.float32),
            pltpu.VMEM((NPR, FCH), jnp.float32),
            pltpu.VMEM((NPR, FCH), jnp.float32),
            pltpu.VMEM((16,), jnp.int32),
            pltpu.SemaphoreType.DMA,
            pltpu.SemaphoreType.DMA,
        ],
    )
    def k(b7_hbm, srcs_hbm, ldst_hbm, tcnt_hbm, out_hbm,
          sv, lv, idxv, rows, st_s, st_q, st_mn, st_mx, ntv, sem0, sem1):
        _segred_body(b7_hbm, srcs_hbm, ldst_hbm, tcnt_hbm, out_hbm,
                     sv, lv, idxv, rows, st_s, st_q, st_mn, st_mx, ntv,
                     (sem0, sem1))
    return k


def _sc_segred(b7, srcs, ldsts, tcnt):
    return _segred_kernel()(b7, srcs, ldsts, tcnt)


def _segred_body(b7_hbm, srcs_hbm, ldst_hbm, tcnt_hbm, out_hbm,
                 sv, lv, idxv, rows, st_s, st_q, st_mn, st_mx, ntv, sems):
    w = _wid()
    pltpu.sync_copy(tcnt_hbm.at[pl.ds(pl.multiple_of(w * 8, 8), 8)],
                    ntv.at[pl.ds(0, 8)])
    nt = ntv[pl.ds(0, 16)][0]
    nch2 = (nt + 2 * SCH - 1) // (2 * SCH)
    lanes = lax.iota(jnp.int32, 16)

    def fpass(k, carry0):
        def zrow(i, carry):
            for j in range(FCH // 16):
                sl = pl.ds(j * 16, 16)
                st_s[i, sl] = jnp.zeros((16,), jnp.float32)
                st_q[i, sl] = jnp.zeros((16,), jnp.float32)
                st_mn[i, sl] = jnp.full((16,), BIGF, jnp.float32)
                st_mx[i, sl] = jnp.full((16,), -BIGF, jnp.float32)
            return carry
        lax.fori_loop(0, NPR, zrow, 0)

        kbase = k * N

        def load_san_fire(c, b):
            off = pl.multiple_of(c * SCH, SCH)
            pltpu.sync_copy(srcs_hbm.at[w, pl.ds(off, SCH)], sv.at[b])
            pltpu.sync_copy(ldst_hbm.at[w, pl.ds(off, SCH)],
                            lv.at[b, pl.ds(0, SCH)])

            def mkidx(j, carry2):
                pos = off + j * 16 + lanes
                valid = pos < nt
                s = jnp.where(valid, sv[b, pl.ds(j * 16, 16)], w)
                idxv[b, j // (GCH // 16),
                     pl.ds((j % (GCH // 16)) * 16, 16)] = s + kbase
                lvec = lv[b, pl.ds(j * 16, 16)]
                lv[b, pl.ds(j * 16, 16)] = jnp.where(valid, lvec, NP)
                return carry2
            lax.fori_loop(0, SCH // 16, mkidx, 0)
            for q in range(SCH // GCH):
                pltpu.async_copy(b7_hbm.at[idxv.at[b, q]],
                                 rows.at[b, pl.ds(q * GCH, GCH)], sems[b])

        def drain(b):
            for q in range(SCH // GCH):
                pltpu.make_async_copy(
                    b7_hbm.at[pl.ds(0, GCH)],
                    rows.at[b, pl.ds(q * GCH, GCH)], sems[b]).wait()

        def compute(b):
            def group(g, carry3):
                base16 = pl.multiple_of(g * 16, 16)
                lvec = lv[b, pl.ds(base16, 16)]
                for j16 in range(16):
                    l = lvec[j16]
                    i = base16 + j16
                    for j in range(FCH // 16):
                        sl = pl.ds(j * 16, 16)
                        v = rows[b, i, sl]
                        plsc.addupdate(st_s.at[l, sl], v)
                        plsc.addupdate(st_q.at[l, sl], v * v)
                        st_mn[l, sl] = jnp.minimum(st_mn[l, sl], v)
                        st_mx[l, sl] = jnp.maximum(st_mx[l, sl], v)
                return carry3
            lax.fori_loop(0, SCH // 16, group, 0)

        load_san_fire(0, 0)

        def pipe(t, carry):
            for b in (0, 1):
                c = 2 * t + b
                load_san_fire(c + 1, (b + 1) % 2)
                drain(b)
                compute(b)
            return carry
        lax.fori_loop(0, nch2, pipe, 0)
        drain(0)

        row0 = pl.multiple_of(w * NPR, 8)
        col = pl.multiple_of(k * FCH, FCH)
        pltpu.sync_copy(st_s, out_hbm.at[0, pl.ds(row0, NPR), pl.ds(col, FCH)])
        pltpu.sync_copy(st_q, out_hbm.at[1, pl.ds(row0, NPR), pl.ds(col, FCH)])
        pltpu.sync_copy(st_mn, out_hbm.at[2, pl.ds(row0, NPR), pl.ds(col, FCH)])
        pltpu.sync_copy(st_mx, out_hbm.at[3, pl.ds(row0, NPR), pl.ds(col, FCH)])
        return carry0

    lax.fori_loop(0, NFP, fpass, 0)


def _tc_mm(h, wcat, bias, bn=None):
    blk = 1000
    grid = (N // blk,)

    def body_plain(h_ref, w_ref, b_ref, a_ref, bp_ref):
        ab = jnp.dot(h_ref[...], w_ref[...], preferred_element_type=jnp.float32)
        a_ref[...] = ab[:, :400] + b_ref[...]
        bp_ref[...] = ab[:, 400:]

    def body_bn(h_ref, w_ref, b_ref, st_ref, g_ref, bb_ref, a_ref, bp_ref,
                h_out_ref):
        mu = st_ref[0:1, :]
        var = st_ref[1:2, :]
        hx = (h_ref[...] - mu) * jax.lax.rsqrt(var + 1e-5)
        hx = jnp.maximum(hx * g_ref[...] + bb_ref[...], 0.0)
        h_out_ref[...] = hx
        ab = jnp.dot(hx, w_ref[...], preferred_element_type=jnp.float32)
        a_ref[...] = ab[:, :400] + b_ref[...]
        bp_ref[...] = ab[:, 400:]

    hspec = pl.BlockSpec((blk, F_IN), lambda i: (i, 0))
    wspec = pl.BlockSpec((F_IN, 848), lambda i: (0, 0))
    bspec = pl.BlockSpec((1, 400), lambda i: (0, 0))
    aspec = pl.BlockSpec((blk, 400), lambda i: (i, 0))
    bpspec = pl.BlockSpec((blk, 448), lambda i: (i, 0))
    if bn is None:
        return pl.pallas_call(
            body_plain, grid=grid, name="tc_mm_plain",
            in_specs=[hspec, wspec, bspec],
            out_specs=[aspec, bpspec],
            out_shape=[jax.ShapeDtypeStruct((N, 400), jnp.float32),
                       jax.ShapeDtypeStruct((N, 448), jnp.float32)],
        )(h, wcat, bias.reshape(1, 400))
    stats, g, bb = bn
    return pl.pallas_call(
        body_bn, grid=grid, name="tc_mm_bn",
        in_specs=[hspec, wspec, bspec,
                  pl.BlockSpec((2, F_IN), lambda i: (0, 0)),
                  pl.BlockSpec((1, F_IN), lambda i: (0, 0)),
                  pl.BlockSpec((1, F_IN), lambda i: (0, 0))],
        out_specs=[aspec, bpspec, pl.BlockSpec((blk, F_IN), lambda i: (i, 0))],
        out_shape=[jax.ShapeDtypeStruct((N, 400), jnp.float32),
                   jax.ShapeDtypeStruct((N, 448), jnp.float32),
                   jax.ShapeDtypeStruct((N, F_IN), jnp.float32)],
    )(h, wcat, bias.reshape(1, 400), stats, g.reshape(1, F_IN),
      bb.reshape(1, F_IN))


def _tc_post(a, ssum, ssq, smn, smx, cnt, x, p0r, wplain, wamp, watt,
             pb_flat, clw, clb):
    blk = 400
    grid = (N // blk,)

    def body(a_ref, s_ref, q_ref, mn_ref, mx_ref, c_ref, x_ref, p0_ref,
             wp_ref, wa_ref, wt_ref, pb_ref, clw_ref, clb_ref,
             y_ref, st_ref, acc):
        step = pl.program_id(0)
        av = a_ref[...]
        cnt_col = c_ref[...]
        deg = jnp.maximum(cnt_col, 1.0)
        has = cnt_col > 0.0
        sb = s_ref[...][:, :400]
        qb = q_ref[...][:, :400]
        mean = (cnt_col * av + sb) / deg
        meansq = (cnt_col * av * av + 2.0 * av * sb + qb) / deg
        std = jnp.sqrt(jnp.maximum(meansq - mean * mean, 0.0) + 1e-5)
        mn = jnp.where(has, av + mn_ref[...][:, :400], 0.0)
        mx = jnp.where(has, av + mx_ref[...][:, :400], 0.0)

        aggc = jnp.concatenate([mean, mn, mx, std], axis=1)

        logd = jnp.log(deg + 1.0)
        amp = logd / AVG_DEG_LOG
        att = AVG_DEG_LOG / logd

        y5 = (jnp.dot(x_ref[...], p0_ref[...],
                      preferred_element_type=jnp.float32)
              + jnp.dot(aggc, wp_ref[...], preferred_element_type=jnp.float32)
              + amp * jnp.dot(aggc, wa_ref[...],
                              preferred_element_type=jnp.float32)
              + att * jnp.dot(aggc, wt_ref[...],
                              preferred_element_type=jnp.float32)
              + pb_ref[...])
        y = jnp.dot(y5, clw_ref[...], preferred_element_type=jnp.float32) \
            + clb_ref[...]
        y_ref[...] = y

        @pl.when(step == 0)
        def _():
            acc[...] = jnp.zeros_like(acc)
        acc[0, :] += jnp.sum(y, axis=0)
        acc[1, :] += jnp.sum(y * y, axis=0)
        st_ref[...] = acc[...]

    return pl.pallas_call(
        body, grid=grid, name="tc_post",
        in_specs=[
            pl.BlockSpec((blk, 400), lambda i: (i, 0)),
            pl.BlockSpec((blk, 448), lambda i: (i, 0)),
            pl.BlockSpec((blk, 448), lambda i: (i, 0)),
            pl.BlockSpec((blk, 448), lambda i: (i, 0)),
            pl.BlockSpec((blk, 448), lambda i: (i, 0)),
            pl.BlockSpec((blk, 1), lambda i: (i, 0)),
            pl.BlockSpec((blk, F_IN), lambda i: (i, 0)),
            pl.BlockSpec((F_IN, 80), lambda i: (0, 0)),
            pl.BlockSpec((1600, 80), lambda i: (0, 0)),
            pl.BlockSpec((1600, 80), lambda i: (0, 0)),
            pl.BlockSpec((1600, 80), lambda i: (0, 0)),
            pl.BlockSpec((1, 80), lambda i: (0, 0)),
            pl.BlockSpec((80, 80), lambda i: (0, 0)),
            pl.BlockSpec((1, 80), lambda i: (0, 0)),
        ],
        out_specs=[pl.BlockSpec((blk, F_IN), lambda i: (i, 0)),
                   pl.BlockSpec((2, F_IN), lambda i: (0, 0))],
        out_shape=[jax.ShapeDtypeStruct((N, F_IN), jnp.float32),
                   jax.ShapeDtypeStruct((2, F_IN), jnp.float32)],
        scratch_shapes=[pltpu.VMEM((2, F_IN), jnp.float32)],
    )(a, ssum, ssq, smn, smx, cnt, x, p0r, wplain, wamp, watt,
      pb_flat.reshape(1, 80), clw, clb.reshape(1, 80))


NPOOL = 10240


def _tc_pool(y2, stats2, g2, b2, batch, lin_w, lin_b):
    blk = 640
    grid = (NPOOL // blk,)

    def body(y_ref, st_ref, g_ref, bb_ref, bt_ref, lw_ref, lb_ref, o_ref,
             pmax, pcnt):
        step = pl.program_id(0)

        @pl.when(step == 0)
        def _():
            pmax[...] = jnp.full_like(pmax, -BIGF)
            pcnt[...] = jnp.zeros_like(pcnt)

        mu = st_ref[0:1, :]
        var = st_ref[1:2, :]
        h = (y_ref[...] - mu) * jax.lax.rsqrt(var + 1e-5)
        h = jnp.maximum(h * g_ref[...] + bb_ref[...], 0.0)

        bt = bt_ref[...]
        for g in range(NUM_GRAPHS):
            maskg = bt == g
            cand = jnp.where(maskg, h, -BIGF)
            mg = jnp.max(cand, axis=0, keepdims=True)
            pmax[g:g + 1, :] = jnp.maximum(pmax[g:g + 1, :], mg)
            pcnt[g:g + 1, :] += jnp.sum(maskg.astype(jnp.float32), axis=0,
                                        keepdims=True)

        @pl.when(step == grid[0] - 1)
        def _():
            pooled = jnp.where(pcnt[...] > 0.0, pmax[...], 0.0)
            o_ref[...] = jnp.dot(pooled, lw_ref[...],
                                 preferred_element_type=jnp.float32) \
                + lb_ref[...]

    return pl.pallas_call(
        body, grid=grid, name="tc_pool",
        in_specs=[
            pl.BlockSpec((blk, F_IN), lambda i: (i, 0)),
            pl.BlockSpec((2, F_IN), lambda i: (0, 0)),
            pl.BlockSpec((1, F_IN), lambda i: (0, 0)),
            pl.BlockSpec((1, F_IN), lambda i: (0, 0)),
            pl.BlockSpec((blk, 1), lambda i: (i, 0)),
            pl.BlockSpec((F_IN, 20), lambda i: (0, 0)),
            pl.BlockSpec((1, 20), lambda i: (0, 0)),
        ],
        out_specs=pl.BlockSpec((NUM_GRAPHS, 20), lambda i: (0, 0)),
        out_shape=jax.ShapeDtypeStruct((NUM_GRAPHS, 20), jnp.float32),
        scratch_shapes=[pltpu.VMEM((NUM_GRAPHS, F_IN), jnp.float32),
                        pltpu.VMEM((NUM_GRAPHS, 1), jnp.float32)],
    )(y2, stats2, g2.reshape(1, F_IN), b2.reshape(1, F_IN), batch,
      lin_w, lin_b.reshape(1, 20))


def _prep_pre(pre_w, pre_b):
    wd = jnp.transpose(pre_w[:, :F_IN, :], (1, 0, 2)).reshape(F_IN, 400)
    ws = jnp.transpose(pre_w[:, F_IN:, :], (1, 0, 2)).reshape(F_IN, 400)
    ws = jnp.pad(ws, ((0, 0), (0, 48)))
    wcat = jnp.concatenate([wd, ws], axis=1)
    return wcat, pre_b.reshape(400)


def _prep_post(post_w):
    p0r = jnp.transpose(post_w[:, :80, :], (1, 0, 2)).reshape(80, 80)
    eye = jnp.eye(TOWERS, dtype=jnp.float32)

    def bd80(sl):
        return (sl[:, :, None, :] * eye[:, None, :, None]).reshape(400, 80)

    def group(base):
        return jnp.concatenate(
            [bd80(post_w[:, base + 80 * a: base + 80 * (a + 1), :])
             for a in range(4)], axis=0)
    wplain = group(80)
    wamp = group(400)
    watt = group(720)
    return p0r, wplain, wamp, watt


def _layer(h_act_or_y, wcat, bias, post_parts, post_b, clw, clb,
           srcs, ldsts, tcnt, cnt_col, bn=None):
    outs = _tc_mm(h_act_or_y, wcat, bias, bn=bn)
    if bn is None:
        a, bp = outs
        x_act = h_act_or_y
    else:
        a, bp, x_act = outs
    b7 = bp.reshape(N, NFP, FCH).transpose(1, 0, 2).reshape(NFP * N, FCH)
    segred = _sc_segred(b7, srcs, ldsts, tcnt)
    segred = segred.reshape(4, NW, NPR, 448)[:, :, :NP, :].reshape(4, NPAD, 448)
    ssum, ssq, smn, smx = (segred[i, :N, :] for i in range(4))
    p0r, wplain, wamp, watt = post_parts
    return _tc_post(a, ssum, ssq, smn, smx, cnt_col, x_act, p0r, wplain,
                    wamp, watt, post_b.reshape(80), clw, clb)


def kernel(x, edge_index, batch, node_emb, pre_w1, pre_b1, post_w1, post_b1,
           conv_lin_w1, conv_lin_b1, bn_g1, bn_b1,
           pre_w2, pre_b2, post_w2, post_b2, conv_lin_w2, conv_lin_b2,
           bn_g2, bn_b2, lin_w, lin_b):
    table16 = jnp.pad(node_emb, ((0, 0), (0, 14)))
    emb16 = _sc_emb(table16, x.reshape(-1))
    h1 = emb16[:, :2].reshape(N, F_IN)

    srcs, ldsts, cntp, tcnt = _sc_csr(edge_index)
    cnt_col = cntp[:N].reshape(N, 1)

    wcat1, bias1 = _prep_pre(pre_w1, pre_b1)
    wcat2, bias2 = _prep_pre(pre_w2, pre_b2)
    parts1 = _prep_post(post_w1)
    parts2 = _prep_post(post_w2)

    y1, stats1 = _layer(h1, wcat1, bias1, parts1, post_b1, conv_lin_w1,
                        conv_lin_b1, srcs, ldsts, tcnt, cnt_col)
    y2, stats2 = _layer(y1, wcat2, bias2, parts2, post_b2, conv_lin_w2,
                        conv_lin_b2, srcs, ldsts, tcnt, cnt_col,
                        bn=(_fix_stats(stats1), bn_g1, bn_b1))
    y2p = jnp.pad(y2, ((0, NPOOL - N), (0, 0)))
    batchp = jnp.pad(batch.reshape(N, 1), ((0, NPOOL - N), (0, 0)),
                     constant_values=127)
    out = _tc_pool(y2p, _fix_stats(stats2), bn_g2, bn_b2, batchp,
                   lin_w, lin_b)
    return out


def _fix_stats(stats_raw):
    mu = stats_raw[0] / float(N)
    var = stats_raw[1] / float(N) - mu * mu
    return jnp.stack([mu, var])


# speedup vs baseline: 41.7229x; 1.1320x over previous
import functools
import numpy as np
import jax
import jax.numpy as jnp
from jax import lax
from jax.experimental import pallas as pl
from jax.experimental.pallas import tpu as pltpu
from jax.experimental.pallas import tpu_sc as plsc

N = 10000
E = 160000
TOWERS = 5
F_IN = 80
F_OUT = 16
NUM_GRAPHS = 64
VOCAB = 10000
AVG_DEG_LOG = float(np.log(17.0))

NW = 32
NP = 320
NPR = NP + 8
NPAD = NW * NP
FCH = 64
NFP = 7
FP = NFP * FCH
ECH = 4000
LCAP = E + 8192
GCH = 128
SCH = 256

@functools.lru_cache(maxsize=None)
def _mesh():
    return plsc.VectorSubcoreMesh(core_axis_name="c", subcore_axis_name="s")


def _wid():
    return lax.axis_index("s") * 2 + lax.axis_index("c")


NIDX = N * 40
NGCH = NIDX // GCH


@functools.lru_cache(maxsize=None)
def _emb_kernel():
    @functools.partial(
        pl.kernel, mesh=_mesh(),
        compiler_params=pltpu.CompilerParams(use_tc_tiling_on_sc=False, needs_layout_passes=False),
        name="sc_emb",
        out_type=jax.ShapeDtypeStruct((NIDX, 16), jnp.float32),
        scratch_types=[
            pltpu.VMEM((GCH,), jnp.int32),
            pltpu.VMEM((GCH, 16), jnp.float32),
            pltpu.SemaphoreType.DMA,
        ],
    )
    def k(table_hbm, idx_hbm, out_hbm, idx_v, rows_v, sem):
        w = _wid()
        nci = (NGCH - w + NW - 1) // NW

        def body(c, carry):
            ci = w + c * NW
            base = pl.multiple_of(ci * GCH, GCH)
            pltpu.sync_copy(idx_hbm.at[pl.ds(base, GCH)], idx_v)
            pltpu.async_copy(table_hbm.at[idx_v], rows_v, sem).wait()
            pltpu.sync_copy(rows_v, out_hbm.at[pl.ds(base, GCH)])
            return carry

        lax.fori_loop(0, nci, body, 0)
    return k


def _sc_emb(table16, idx):
    return _emb_kernel()(table16, idx)


@functools.lru_cache(maxsize=None)
def _csr_kernel():
    @functools.partial(
        pl.kernel, mesh=_mesh(),
        compiler_params=pltpu.CompilerParams(use_tc_tiling_on_sc=False, needs_layout_passes=False),
        name="sc_csr",
        out_type=[
            jax.ShapeDtypeStruct((NW, LCAP), jnp.int32),
            jax.ShapeDtypeStruct((NW, LCAP), jnp.int32),
            jax.ShapeDtypeStruct((NPAD,), jnp.float32),
            jax.ShapeDtypeStruct((NW * 8,), jnp.int32),
        ],
        scratch_types=[
            pltpu.VMEM((2, ECH), jnp.int32),
            pltpu.VMEM((2, ECH), jnp.int32),
            pltpu.VMEM((ECH + 16,), jnp.int32),
            pltpu.VMEM((ECH + 16,), jnp.int32),
            pltpu.VMEM((NP + 16,), jnp.float32),
            pltpu.VMEM((16,), jnp.int32),
            pltpu.SemaphoreType.DMA,
            pltpu.SemaphoreType.DMA,
        ],
    )
    def k(edge_hbm, srcs_hbm, ldst_hbm, cnt_hbm, tcnt_hbm,
          sv, dv, csv, clv, cntv, tmpv, sem0, sem1):
        _csr_body(edge_hbm, srcs_hbm, ldst_hbm, cnt_hbm, tcnt_hbm,
                  sv, dv, csv, clv, cntv, tmpv, (sem0, sem1))
    return k


def _sc_csr(edge_index):
    return _csr_kernel()(edge_index)


def _csr_body(edge_hbm, srcs_hbm, ldst_hbm, cnt_hbm, tcnt_hbm,
              sv, dv, csv, clv, cntv, tmpv, sems):
    w = _wid()
    lo = pl.multiple_of(w * NP, NP)

    def zbody(i, carry):
        cntv[pl.ds(i * 16, 16)] = jnp.zeros((16,), jnp.float32)
        return carry
    lax.fori_loop(0, (NP + 16) // 16, zbody, 0)

    lanes = lax.iota(jnp.int32, 16)
    filler_ld = NP + (lanes % 8)

    def fire(c, b):
        off = pl.multiple_of(jnp.minimum(c * ECH, E - ECH), 16)
        pltpu.async_copy(edge_hbm.at[0, pl.ds(off, ECH)], sv.at[b], sems[b])
        pltpu.async_copy(edge_hbm.at[1, pl.ds(off, ECH)], dv.at[b], sems[b])

    def drain(b):
        pltpu.make_async_copy(edge_hbm.at[0, pl.ds(0, ECH)], sv.at[b],
                              sems[b]).wait()
        pltpu.make_async_copy(edge_hbm.at[1, pl.ds(0, ECH)], dv.at[b],
                              sems[b]).wait()

    def process(b, w_off):
        def step(j, cur):
            d = dv[b, pl.ds(j * 16, 16)]
            s = sv[b, pl.ds(j * 16, 16)]
            m = (d >= lo) & (d < lo + NP)
            ld = jnp.where(m, d - lo, NP)
            rank = plsc.cumsum(jnp.where(m, 1, 0).astype(jnp.int32))
            pos = jnp.where(m, cur + rank - 1, cur + 15)
            plsc.store_scatter(csv, [pos], s)
            plsc.store_scatter(clv, [pos], ld)
            plsc.addupdate_scatter(cntv, [ld], jnp.ones((16,), jnp.float32))
            return cur + rank[15]

        cur = lax.fori_loop(0, ECH // 16, step, 0)
        csv[pl.ds(cur, 16)] = jnp.full((16,), w, jnp.int32)
        clv[pl.ds(cur, 16)] = filler_ld
        cur_pad = ((cur + 15) // 16) * 16
        w_off = pl.multiple_of(w_off, 16)
        pltpu.sync_copy(csv, srcs_hbm.at[w, pl.ds(w_off, ECH + 16)])
        pltpu.sync_copy(clv, ldst_hbm.at[w, pl.ds(w_off, ECH + 16)])
        return w_off + cur_pad

    fire(0, 0)

    def outer(t, w_off):
        for b in (0, 1):
            c = 2 * t + b
            fire(c + 1, (b + 1) % 2)
            drain(b)
            w_off = process(b, w_off)
        return w_off

    total = lax.fori_loop(0, (E // ECH) // 2, outer, 0)
    drain(0)
    tmpv[...] = jnp.broadcast_to(total, (16,)).astype(jnp.int32)
    pltpu.sync_copy(tmpv.at[pl.ds(0, 8)],
                    tcnt_hbm.at[pl.ds(pl.multiple_of(w * 8, 8), 8)])
    pltpu.sync_copy(cntv.at[pl.ds(0, NP)], cnt_hbm.at[pl.ds(lo, NP)])


BIGF = 3.0e38


@functools.lru_cache(maxsize=None)
def _segred_kernel():
    @functools.partial(
        pl.kernel, mesh=_mesh(),
        compiler_params=pltpu.CompilerParams(use_tc_tiling_on_sc=False, needs_layout_passes=False),
        name="sc_segred",
        out_type=[jax.ShapeDtypeStruct((NW * NP, FP), jnp.float32)] * 4,
        scratch_types=[
            pltpu.VMEM((2, SCH), jnp.int32),
            pltpu.VMEM((2, SCH + 16), jnp.int32),
            pltpu.VMEM((2, SCH // GCH, GCH), jnp.int32),
            pltpu.VMEM((2, SCH, FCH), jnp.float32),
            pltpu.VMEM((NPR, FCH), jnp.float32),
            pltpu.VMEM((NPR, FCH), jnp.float32),
            pltpu.VMEM((NPR, FCH), jnp.float32),
            pltpu.VMEM((NPR, FCH), jnp.float32),
            pltpu.VMEM((16,), jnp.int32),
            pltpu.SemaphoreType.DMA,
            pltpu.SemaphoreType.DMA,
        ],
    )
    def k(b0, b1, b2, b3, b4, b5, b6, srcs_hbm, ldst_hbm, tcnt_hbm,
          o0, o1, o2, o3,
          sv, lv, idxv, rows, st_s, st_q, st_mn, st_mx, ntv, sem0, sem1):
        _segred_body((b0, b1, b2, b3, b4, b5, b6), srcs_hbm, ldst_hbm,
                     tcnt_hbm, (o0, o1, o2, o3), sv, lv, idxv, rows,
                     st_s, st_q, st_mn, st_mx, ntv, (sem0, sem1))
    return k


def _sc_segred(bs, srcs, ldsts, tcnt):
    return _segred_kernel()(*bs, srcs, ldsts, tcnt)


def _segred_body(btabs, srcs_hbm, ldst_hbm, tcnt_hbm, outs,
                 sv, lv, idxv, rows, st_s, st_q, st_mn, st_mx, ntv, sems):
    w = _wid()
    pltpu.sync_copy(tcnt_hbm.at[pl.ds(pl.multiple_of(w * 8, 8), 8)],
                    ntv.at[pl.ds(0, 8)])
    nt = ntv[pl.ds(0, 16)][0]
    nch2 = (nt + 2 * SCH - 1) // (2 * SCH)
    lanes = lax.iota(jnp.int32, 16)

    def fpass(k, b7_hbm):
        def zrow(i, carry):
            for j in range(FCH // 16):
                sl = pl.ds(j * 16, 16)
                st_s[i, sl] = jnp.zeros((16,), jnp.float32)
                st_q[i, sl] = jnp.zeros((16,), jnp.float32)
                st_mn[i, sl] = jnp.full((16,), BIGF, jnp.float32)
                st_mx[i, sl] = jnp.full((16,), -BIGF, jnp.float32)
            return carry
        lax.fori_loop(0, NPR, zrow, 0)

        def load_san_fire(c, b):
            off = pl.multiple_of(c * SCH, SCH)
            pltpu.sync_copy(srcs_hbm.at[w, pl.ds(off, SCH)], sv.at[b])
            pltpu.sync_copy(ldst_hbm.at[w, pl.ds(off, SCH)],
                            lv.at[b, pl.ds(0, SCH)])

            def mkidx(j, carry2):
                pos = off + j * 16 + lanes
                valid = pos < nt
                s = jnp.where(valid, sv[b, pl.ds(j * 16, 16)], w)
                idxv[b, j // (GCH // 16),
                     pl.ds((j % (GCH // 16)) * 16, 16)] = s
                lvec = lv[b, pl.ds(j * 16, 16)]
                lv[b, pl.ds(j * 16, 16)] = jnp.where(valid, lvec, NP)
                return carry2
            lax.fori_loop(0, SCH // 16, mkidx, 0)
            for q in range(SCH // GCH):
                pltpu.async_copy(b7_hbm.at[idxv.at[b, q]],
                                 rows.at[b, pl.ds(q * GCH, GCH)], sems[b])

        def drain(b):
            for q in range(SCH // GCH):
                pltpu.make_async_copy(
                    b7_hbm.at[pl.ds(0, GCH)],
                    rows.at[b, pl.ds(q * GCH, GCH)], sems[b]).wait()

        def compute(b):
            def group(g, carry3):
                base16 = pl.multiple_of(g * 16, 16)
                lvec = lv[b, pl.ds(base16, 16)]
                for j16 in range(16):
                    l = lvec[j16]
                    i = base16 + j16
                    for j in range(FCH // 16):
                        sl = pl.ds(j * 16, 16)
                        v = rows[b, i, sl]
                        plsc.addupdate(st_s.at[l, sl], v)
                        plsc.addupdate(st_q.at[l, sl], v * v)
                        st_mn[l, sl] = jnp.minimum(st_mn[l, sl], v)
                        st_mx[l, sl] = jnp.maximum(st_mx[l, sl], v)
                return carry3
            lax.fori_loop(0, SCH // 16, group, 0)

        load_san_fire(0, 0)

        def pipe(t, carry):
            for b in (0, 1):
                c = 2 * t + b
                load_san_fire(c + 1, (b + 1) % 2)
                drain(b)
                compute(b)
            return carry
        lax.fori_loop(0, nch2, pipe, 0)
        drain(0)

        row0 = pl.multiple_of(w * NP, 8)
        col = k * FCH
        for a, st in enumerate((st_s, st_q, st_mn, st_mx)):
            pltpu.sync_copy(st.at[pl.ds(0, NP)],
                            outs[a].at[pl.ds(row0, NP), pl.ds(col, FCH)])

    for k in range(NFP):
        fpass(k, btabs[k])


def _tc_mm(h, wcat, bias, bn=None):
    blk = 1000
    grid = (N // blk,)

    def write_ab(ab, b_ref, a_ref, bp_refs):
        a_ref[...] = ab[:, :400] + b_ref[...]
        for p in range(NFP):
            bp_refs[p][...] = ab[:, 400 + FCH * p: 400 + FCH * (p + 1)]

    def body_plain(h_ref, w_ref, b_ref, a_ref, *bp_refs):
        ab = jnp.dot(h_ref[...], w_ref[...], preferred_element_type=jnp.float32)
        write_ab(ab, b_ref, a_ref, bp_refs)

    def body_bn(h_ref, w_ref, b_ref, st_ref, g_ref, bb_ref, a_ref,
                h_out_ref, *bp_refs):
        mu = st_ref[0:1, :]
        var = st_ref[1:2, :]
        hx = (h_ref[...] - mu) * jax.lax.rsqrt(var + 1e-5)
        hx = jnp.maximum(hx * g_ref[...] + bb_ref[...], 0.0)
        h_out_ref[...] = hx
        ab = jnp.dot(hx, w_ref[...], preferred_element_type=jnp.float32)
        write_ab(ab, b_ref, a_ref, bp_refs)

    hspec = pl.BlockSpec((blk, F_IN), lambda i: (i, 0))
    wspec = pl.BlockSpec((F_IN, 848), lambda i: (0, 0))
    bspec = pl.BlockSpec((1, 400), lambda i: (0, 0))
    aspec = pl.BlockSpec((blk, 400), lambda i: (i, 0))
    bpspec = pl.BlockSpec((blk, FCH), lambda i: (i, 0))
    bpshapes = [jax.ShapeDtypeStruct((N, FCH), jnp.float32)] * NFP
    if bn is None:
        outs = pl.pallas_call(
            body_plain, grid=grid, name="tc_mm_plain",
            in_specs=[hspec, wspec, bspec],
            out_specs=[aspec] + [bpspec] * NFP,
            out_shape=[jax.ShapeDtypeStruct((N, 400), jnp.float32)] + bpshapes,
        )(h, wcat, bias.reshape(1, 400))
        return outs[0], outs[1:]
    stats, g, bb = bn
    outs = pl.pallas_call(
        body_bn, grid=grid, name="tc_mm_bn",
        in_specs=[hspec, wspec, bspec,
                  pl.BlockSpec((2, F_IN), lambda i: (0, 0)),
                  pl.BlockSpec((1, F_IN), lambda i: (0, 0)),
                  pl.BlockSpec((1, F_IN), lambda i: (0, 0))],
        out_specs=[aspec, pl.BlockSpec((blk, F_IN), lambda i: (i, 0))]
        + [bpspec] * NFP,
        out_shape=[jax.ShapeDtypeStruct((N, 400), jnp.float32),
                   jax.ShapeDtypeStruct((N, F_IN), jnp.float32)] + bpshapes,
    )(h, wcat, bias.reshape(1, 400), stats, g.reshape(1, F_IN),
      bb.reshape(1, F_IN))
    return outs[0], outs[2:], outs[1]


def _tc_post(a, ssum, ssq, smn, smx, cnt, x, p0r, wplain, wamp, watt,
             pb_flat, clw, clb):
    blk = 400
    grid = (N // blk,)

    def body(a_ref, s_ref, q_ref, mn_ref, mx_ref, c_ref, x_ref, p0_ref,
             wp_ref, wa_ref, wt_ref, pb_ref, clw_ref, clb_ref,
             y_ref, st_ref, acc):
        step = pl.program_id(0)
        av = a_ref[...]
        cnt_col = c_ref[...]
        deg = jnp.maximum(cnt_col, 1.0)
        has = cnt_col > 0.0
        sb = s_ref[...][:, :400]
        qb = q_ref[...][:, :400]
        mean = (cnt_col * av + sb) / deg
        meansq = (cnt_col * av * av + 2.0 * av * sb + qb) / deg
        std = jnp.sqrt(jnp.maximum(meansq - mean * mean, 0.0) + 1e-5)
        mn = jnp.where(has, av + mn_ref[...][:, :400], 0.0)
        mx = jnp.where(has, av + mx_ref[...][:, :400], 0.0)

        aggc = jnp.concatenate([mean, mn, mx, std], axis=1)

        logd = jnp.log(deg + 1.0)
        amp = logd / AVG_DEG_LOG
        att = AVG_DEG_LOG / logd

        y5 = (jnp.dot(x_ref[...], p0_ref[...],
                      preferred_element_type=jnp.float32)
              + jnp.dot(aggc, wp_ref[...], preferred_element_type=jnp.float32)
              + amp * jnp.dot(aggc, wa_ref[...],
                              preferred_element_type=jnp.float32)
              + att * jnp.dot(aggc, wt_ref[...],
                              preferred_element_type=jnp.float32)
              + pb_ref[...])
        y = jnp.dot(y5, clw_ref[...], preferred_element_type=jnp.float32) \
            + clb_ref[...]
        y_ref[...] = y

        @pl.when(step == 0)
        def _():
            acc[...] = jnp.zeros_like(acc)
        acc[0, :] += jnp.sum(y, axis=0)
        acc[1, :] += jnp.sum(y * y, axis=0)
        st_ref[...] = acc[...]

    return pl.pallas_call(
        body, grid=grid, name="tc_post",
        in_specs=[
            pl.BlockSpec((blk, 400), lambda i: (i, 0)),
            pl.BlockSpec((blk, 448), lambda i: (i, 0)),
            pl.BlockSpec((blk, 448), lambda i: (i, 0)),
            pl.BlockSpec((blk, 448), lambda i: (i, 0)),
            pl.BlockSpec((blk, 448), lambda i: (i, 0)),
            pl.BlockSpec((blk, 1), lambda i: (i, 0)),
            pl.BlockSpec((blk, F_IN), lambda i: (i, 0)),
            pl.BlockSpec((F_IN, 80), lambda i: (0, 0)),
            pl.BlockSpec((1600, 80), lambda i: (0, 0)),
            pl.BlockSpec((1600, 80), lambda i: (0, 0)),
            pl.BlockSpec((1600, 80), lambda i: (0, 0)),
            pl.BlockSpec((1, 80), lambda i: (0, 0)),
            pl.BlockSpec((80, 80), lambda i: (0, 0)),
            pl.BlockSpec((1, 80), lambda i: (0, 0)),
        ],
        out_specs=[pl.BlockSpec((blk, F_IN), lambda i: (i, 0)),
                   pl.BlockSpec((2, F_IN), lambda i: (0, 0))],
        out_shape=[jax.ShapeDtypeStruct((N, F_IN), jnp.float32),
                   jax.ShapeDtypeStruct((2, F_IN), jnp.float32)],
        scratch_shapes=[pltpu.VMEM((2, F_IN), jnp.float32)],
    )(a, ssum, ssq, smn, smx, cnt, x, p0r, wplain, wamp, watt,
      pb_flat.reshape(1, 80), clw, clb.reshape(1, 80))


NPOOL = 10240


def _tc_pool(y2, stats2, g2, b2, batch, lin_w, lin_b):
    blk = 640
    grid = (NPOOL // blk,)

    def body(y_ref, st_ref, g_ref, bb_ref, bt_ref, lw_ref, lb_ref, o_ref,
             pmax, pcnt):
        step = pl.program_id(0)

        @pl.when(step == 0)
        def _():
            pmax[...] = jnp.full_like(pmax, -BIGF)
            pcnt[...] = jnp.zeros_like(pcnt)

        mu = st_ref[0:1, :]
        var = st_ref[1:2, :]
        h = (y_ref[...] - mu) * jax.lax.rsqrt(var + 1e-5)
        h = jnp.maximum(h * g_ref[...] + bb_ref[...], 0.0)

        bt = bt_ref[...]
        for g in range(NUM_GRAPHS):
            maskg = bt == g
            cand = jnp.where(maskg, h, -BIGF)
            mg = jnp.max(cand, axis=0, keepdims=True)
            pmax[g:g + 1, :] = jnp.maximum(pmax[g:g + 1, :], mg)
            pcnt[g:g + 1, :] += jnp.sum(maskg.astype(jnp.float32), axis=0,
                                        keepdims=True)

        @pl.when(step == grid[0] - 1)
        def _():
            pooled = jnp.where(pcnt[...] > 0.0, pmax[...], 0.0)
            o_ref[...] = jnp.dot(pooled, lw_ref[...],
                                 preferred_element_type=jnp.float32) \
                + lb_ref[...]

    return pl.pallas_call(
        body, grid=grid, name="tc_pool",
        in_specs=[
            pl.BlockSpec((blk, F_IN), lambda i: (i, 0)),
            pl.BlockSpec((2, F_IN), lambda i: (0, 0)),
            pl.BlockSpec((1, F_IN), lambda i: (0, 0)),
            pl.BlockSpec((1, F_IN), lambda i: (0, 0)),
            pl.BlockSpec((blk, 1), lambda i: (i, 0)),
            pl.BlockSpec((F_IN, 20), lambda i: (0, 0)),
            pl.BlockSpec((1, 20), lambda i: (0, 0)),
        ],
        out_specs=pl.BlockSpec((NUM_GRAPHS, 20), lambda i: (0, 0)),
        out_shape=jax.ShapeDtypeStruct((NUM_GRAPHS, 20), jnp.float32),
        scratch_shapes=[pltpu.VMEM((NUM_GRAPHS, F_IN), jnp.float32),
                        pltpu.VMEM((NUM_GRAPHS, 1), jnp.float32)],
    )(y2, stats2, g2.reshape(1, F_IN), b2.reshape(1, F_IN), batch,
      lin_w, lin_b.reshape(1, 20))


def _prep_pre(pre_w, pre_b):
    wd = jnp.transpose(pre_w[:, :F_IN, :], (1, 0, 2)).reshape(F_IN, 400)
    ws = jnp.transpose(pre_w[:, F_IN:, :], (1, 0, 2)).reshape(F_IN, 400)
    ws = jnp.pad(ws, ((0, 0), (0, 48)))
    wcat = jnp.concatenate([wd, ws], axis=1)
    return wcat, pre_b.reshape(400)


def _prep_post(post_w):
    p0r = jnp.transpose(post_w[:, :80, :], (1, 0, 2)).reshape(80, 80)
    eye = jnp.eye(TOWERS, dtype=jnp.float32)

    def bd80(sl):
        return (sl[:, :, None, :] * eye[:, None, :, None]).reshape(400, 80)

    def group(base):
        return jnp.concatenate(
            [bd80(post_w[:, base + 80 * a: base + 80 * (a + 1), :])
             for a in range(4)], axis=0)
    wplain = group(80)
    wamp = group(400)
    watt = group(720)
    return p0r, wplain, wamp, watt


def _layer(h_act_or_y, wcat, bias, post_parts, post_b, clw, clb,
           srcs, ldsts, tcnt, cnt_col, bn=None):
    outs = _tc_mm(h_act_or_y, wcat, bias, bn=bn)
    if bn is None:
        a, bs = outs
        x_act = h_act_or_y
    else:
        a, bs, x_act = outs
    ssum, ssq, smn, smx = _sc_segred(bs, srcs, ldsts, tcnt)
    p0r, wplain, wamp, watt = post_parts
    return _tc_post(a, ssum, ssq, smn, smx, cnt_col, x_act, p0r, wplain,
                    wamp, watt, post_b.reshape(80), clw, clb)


def kernel(x, edge_index, batch, node_emb, pre_w1, pre_b1, post_w1, post_b1,
           conv_lin_w1, conv_lin_b1, bn_g1, bn_b1,
           pre_w2, pre_b2, post_w2, post_b2, conv_lin_w2, conv_lin_b2,
           bn_g2, bn_b2, lin_w, lin_b):
    table16 = jnp.pad(node_emb, ((0, 0), (0, 14)))
    emb16 = _sc_emb(table16, x.reshape(-1))
    h1 = emb16[:, :2].reshape(N, F_IN)

    srcs, ldsts, cntp, tcnt = _sc_csr(edge_index)
    cnt_col = cntp.reshape(NPAD, 1)

    wcat1, bias1 = _prep_pre(pre_w1, pre_b1)
    wcat2, bias2 = _prep_pre(pre_w2, pre_b2)
    parts1 = _prep_post(post_w1)
    parts2 = _prep_post(post_w2)

    y1, stats1 = _layer(h1, wcat1, bias1, parts1, post_b1, conv_lin_w1,
                        conv_lin_b1, srcs, ldsts, tcnt, cnt_col)
    y2, stats2 = _layer(y1, wcat2, bias2, parts2, post_b2, conv_lin_w2,
                        conv_lin_b2, srcs, ldsts, tcnt, cnt_col,
                        bn=(_fix_stats(stats1), bn_g1, bn_b1))
    y2p = jnp.pad(y2, ((0, NPOOL - N), (0, 0)))
    batchp = jnp.pad(batch.reshape(N, 1), ((0, NPOOL - N), (0, 0)),
                     constant_values=127)
    out = _tc_pool(y2p, _fix_stats(stats2), bn_g2, bn_b2, batchp,
                   lin_w, lin_b)
    return out


def _fix_stats(stats_raw):
    mu = stats_raw[0] / float(N)
    var = stats_raw[1] / float(N) - mu * mu
    return jnp.stack([mu, var])
